# Initial kernel scaffold; baseline (speedup 1.0000x reference)
import jax
import jax.numpy as jnp
from jax.experimental import pallas as pl


def kernel(x, edge_index, W1, a_src1, a_dst1, b1, W2, a_src2, a_dst2, b2):
    raise NotImplementedError("write your pallas kernel here")


# speedup vs baseline: 31.4741x; 31.4741x over previous
import functools

import jax
import jax.numpy as jnp
from jax import lax
from jax.experimental import pallas as pl
from jax.experimental.pallas import tpu as pltpu
from jax.experimental.pallas import tpu_sc as plsc

N = 10000
D_IN = 128
HID = 8
HEADS = 8
D_OUT = 128

NPAD = 10240
NC, NS = 2, 16
NW = NC * NS
RPT = NPAD // NS
B = 128

W1ROW = 80
A1ROW = 16


def _tc_a(x_ref, w_ref, wa_ref, hs_ref, ad_ref):
    x = x_ref[...]
    hs_ref[...] = jnp.dot(x, w_ref[...], preferred_element_type=jnp.float32)
    ad_ref[...] = jnp.dot(x, wa_ref[...], preferred_element_type=jnp.float32)


def _tc_b(m0_ref, m1_ref, s0_ref, s1_ref, r8_ref, w2_ref, waa_ref, b1_ref,
          hs2_ref, aa_ref):
    srep = jnp.dot(s0_ref[...] + s1_ref[...], r8_ref[...],
                   preferred_element_type=jnp.float32)
    g = (m0_ref[...] + m1_ref[...]) / (srep + 1e-16) + b1_ref[...]
    g = jnp.where(g > 0, g, jnp.exp(g) - 1.0)
    h2 = jnp.dot(g, w2_ref[...], preferred_element_type=jnp.float32)
    hs2_ref[pl.ds(0, NPAD), :] = h2[:, 0:64]
    hs2_ref[pl.ds(NPAD, NPAD), :] = h2[:, 64:128]
    aa_ref[...] = jnp.dot(g, waa_ref[...], preferred_element_type=jnp.float32)


def _tc_c(m0_ref, m1_ref, sp_ref, ones_ref, b2_ref, out_ref):
    s = jnp.dot(sp_ref[...], ones_ref[...],
                preferred_element_type=jnp.float32)
    v = (jnp.concatenate([m0_ref[...], m1_ref[...]], axis=1) / (s + 1e-16)
         + b2_ref[...])
    m = jnp.max(v, axis=1, keepdims=True)
    u = v - m
    lse = jnp.log(jnp.sum(jnp.exp(u), axis=1, keepdims=True))
    out_ref[...] = u - lse


def _sc_edge1(nblk, src_h, dst_h, hs_h, ad_h, zm_h, zs_h, msg_o, s_o,
              sidx, didx, hsr, adr, outr, exr, exbuf, accm, accs, sem0, sem1):
    cid = lax.axis_index("c")
    sid = lax.axis_index("s")
    wid = sid * NC + cid
    pltpu.sync_copy(zm_h.at[pl.ds(sid * RPT, RPT)],
                    accm.at[pl.ds(sid * RPT, RPT)])
    pltpu.sync_copy(zs_h.at[pl.ds(sid * RPT, RPT)],
                    accs.at[pl.ds(sid * RPT, RPT)])
    plsc.subcore_barrier()
    base = wid * (nblk * B)
    lane = lax.broadcasted_iota(jnp.int32, (16,), 0)
    half = lane // 8
    zero16 = lane // 16
    cols = [half + 2 * k for k in range(4)]
    lane7 = jnp.bitwise_and(lane, 7)
    m8 = lane < 8

    def blk(i, carry):
        off = base + i * B
        pltpu.sync_copy(src_h.at[pl.ds(off, B)], sidx)
        pltpu.sync_copy(dst_h.at[pl.ds(off, B)], didx)
        cpa = pltpu.async_copy(hs_h.at[sidx], hsr, sem0)
        cpb = pltpu.async_copy(ad_h.at[didx], adr, sem1)
        cpa.wait()
        cpb.wait()

        def edge(e, c2):
            asv = hsr[e, pl.ds(64, 16)]
            adv = adr[e, :]
            ev = asv + adv
            ev = jnp.where(ev > 0, ev, 0.2 * ev)
            ex = jnp.exp(ev)
            exbuf[...] = ex
            rowi = zero16 + e
            plsc.store_scatter(exr, [rowi, lane7], ex, mask=m8)
            for k in range(4):
                exb = plsc.load_gather(exbuf, [cols[k]])
                outr[e, pl.ds(16 * k, 16)] = hsr[e, pl.ds(16 * k, 16)] * exb
            return c2

        lax.fori_loop(0, B, edge, 0)
        pltpu.sync_copy(outr, accm.at[didx], add=True)
        pltpu.sync_copy(exr, accs.at[didx], add=True)
        return carry

    lax.fori_loop(0, nblk, blk, 0)
    plsc.subcore_barrier()
    pltpu.sync_copy(accm.at[pl.ds(sid * RPT, RPT)],
                    msg_o.at[cid].at[pl.ds(sid * RPT, RPT)])
    pltpu.sync_copy(accs.at[pl.ds(sid * RPT, RPT)],
                    s_o.at[cid].at[pl.ds(sid * RPT, RPT)])


def _sc_edge2(nblk, src_h, dst_h, hs2_h, as2_h, ad2_h, zm_h, zs_h, msg_o, s_o,
              sidx, didx, sidx2, h2r, outr, astab, adtab, exbuf, s2loc, accm,
              sem0):
    cid = lax.axis_index("c")
    sid = lax.axis_index("s")
    pltpu.sync_copy(as2_h, astab)
    pltpu.sync_copy(ad2_h, adtab)
    pltpu.sync_copy(zs_h, s2loc)
    pltpu.sync_copy(zm_h.at[pl.ds(sid * RPT, RPT)],
                    accm.at[pl.ds(sid * RPT, RPT)])
    plsc.subcore_barrier()
    base = sid * (nblk * B)
    lane = lax.broadcasted_iota(jnp.int32, (16,), 0)
    zero16 = lane // 16
    jcols = [zero16 + j for j in range(16)]
    m1 = lane < 1
    coff = cid * NPAD

    def blk(i, carry):
        off = base + i * B
        pltpu.sync_copy(src_h.at[pl.ds(off, B)], sidx)
        pltpu.sync_copy(dst_h.at[pl.ds(off, B)], didx)

        def sh(g, c2):
            sidx2[pl.ds(g * 16, 16)] = sidx[pl.ds(g * 16, 16)] + coff
            return c2

        lax.fori_loop(0, B // 16, sh, 0)
        pltpu.async_copy(hs2_h.at[sidx2], h2r, sem0).wait()

        def grp(g, c2):
            s16 = sidx[pl.ds(g * 16, 16)]
            d16 = didx[pl.ds(g * 16, 16)]
            asg = plsc.load_gather(astab, [s16])
            adg = plsc.load_gather(adtab, [d16])
            ev = asg + adg
            ev = jnp.where(ev > 0, ev, 0.2 * ev)
            ex16 = jnp.exp(ev)
            exbuf[...] = ex16
            for j in range(16):
                e = g * 16 + j
                exb = plsc.load_gather(exbuf, [jcols[j]])
                dv = plsc.load_gather(didx, [zero16 + e])
                plsc.addupdate_scatter(s2loc, [dv], exb, mask=m1)
                for k in range(4):
                    outr[e, pl.ds(16 * k, 16)] = (
                        h2r[e, pl.ds(16 * k, 16)] * exb)
            return c2

        lax.fori_loop(0, B // 16, grp, 0)
        pltpu.sync_copy(outr, accm.at[didx], add=True)
        return carry

    lax.fori_loop(0, nblk, blk, 0)
    plsc.subcore_barrier()
    pltpu.sync_copy(accm.at[pl.ds(sid * RPT, RPT)],
                    msg_o.at[cid].at[pl.ds(sid * RPT, RPT)])
    pltpu.sync_copy(s2loc, s_o.at[cid].at[sid])


def kernel(x, edge_index, W1, a_src1, a_dst1, b1, W2, a_src2, a_dst2, b2):
    f32 = jnp.float32
    i32 = jnp.int32

    e_real = edge_index.shape[1] + N
    nblk = -(-e_real // (NW * B))
    e_pad = NW * B * nblk
    loop = jnp.arange(N, dtype=i32)
    pad = jnp.full((e_pad - e_real,), N, dtype=i32)
    src = jnp.concatenate([edge_index[0].astype(i32), loop, pad])
    dst = jnp.concatenate([edge_index[1].astype(i32), loop, pad])

    k64 = jnp.arange(HEADS * HID)
    as64 = jnp.zeros((HEADS * HID, HEADS), f32).at[k64, k64 // HID].set(
        a_src1.reshape(-1))
    ad64 = jnp.zeros((HEADS * HID, HEADS), f32).at[k64, k64 // HID].set(
        a_dst1.reshape(-1))
    r8 = jnp.zeros((HEADS, HEADS * HID), f32).at[k64 // HID, k64].set(1.0)
    w1cat = jnp.concatenate(
        [W1, W1 @ as64, jnp.zeros((D_IN, 8), f32)], axis=1)
    w1ad = jnp.concatenate(
        [W1 @ ad64, jnp.zeros((D_IN, 8), f32)], axis=1)
    waa = jnp.concatenate(
        [W2 @ a_src2.reshape(-1, 1), W2 @ a_dst2.reshape(-1, 1),
         jnp.zeros((HEADS * HID, 14), f32)], axis=1)
    ones16 = jnp.ones((NS, 1), f32)

    x_pad = jnp.zeros((NPAD, D_IN), f32).at[:N].set(x)

    hs1, ad1 = pl.pallas_call(
        _tc_a,
        out_shape=(jax.ShapeDtypeStruct((NPAD, W1ROW), f32),
                   jax.ShapeDtypeStruct((NPAD, A1ROW), f32)),
    )(x_pad, w1cat, w1ad)

    zm1 = jnp.zeros((NPAD, 64), f32)
    zs1 = jnp.zeros((NPAD, 8), f32)
    mesh = plsc.VectorSubcoreMesh(core_axis_name="c", subcore_axis_name="s")
    sc_params = pltpu.CompilerParams(
        use_tc_tiling_on_sc=False, needs_layout_passes=False)
    msg1, s1 = pl.kernel(
        functools.partial(_sc_edge1, nblk),
        out_type=(jax.ShapeDtypeStruct((NC, NPAD, 64), f32),
                  jax.ShapeDtypeStruct((NC, NPAD, 8), f32)),
        mesh=mesh,
        compiler_params=sc_params,
        scratch_types=[
            pltpu.VMEM((B,), i32),
            pltpu.VMEM((B,), i32),
            pltpu.VMEM((B, W1ROW), f32),
            pltpu.VMEM((B, A1ROW), f32),
            pltpu.VMEM((B, 64), f32),
            pltpu.VMEM((B, 8), f32),
            pltpu.VMEM((16,), f32),
            pltpu.VMEM_SHARED((NPAD, 64), f32),
            pltpu.VMEM_SHARED((NPAD, 8), f32),
            pltpu.SemaphoreType.DMA,
            pltpu.SemaphoreType.DMA,
        ],
    )(src, dst, hs1, ad1, zm1, zs1)

    hs2, aa = pl.pallas_call(
        _tc_b,
        out_shape=(jax.ShapeDtypeStruct((2 * NPAD, 64), f32),
                   jax.ShapeDtypeStruct((NPAD, 16), f32)),
    )(msg1[0], msg1[1], s1[0], s1[1], r8, W2, waa, b1.reshape(1, -1))

    as2 = aa[:, 0].reshape(NPAD)
    ad2 = aa[:, 1].reshape(NPAD)

    nblk2 = e_pad // (NS * B)
    zm2 = jnp.zeros((NPAD, 64), f32)
    zs2 = jnp.zeros((NPAD,), f32)
    msg2, s2 = pl.kernel(
        functools.partial(_sc_edge2, nblk2),
        out_type=(jax.ShapeDtypeStruct((NC, NPAD, 64), f32),
                  jax.ShapeDtypeStruct((NC, NS, NPAD), f32)),
        mesh=mesh,
        compiler_params=sc_params,
        scratch_types=[
            pltpu.VMEM((B,), i32),
            pltpu.VMEM((B,), i32),
            pltpu.VMEM((B,), i32),
            pltpu.VMEM((B, 64), f32),
            pltpu.VMEM((B, 64), f32),
            pltpu.VMEM((NPAD,), f32),
            pltpu.VMEM((NPAD,), f32),
            pltpu.VMEM((16,), f32),
            pltpu.VMEM((NPAD,), f32),
            pltpu.VMEM_SHARED((NPAD, 64), f32),
            pltpu.SemaphoreType.DMA,
        ],
    )(src, dst, hs2, as2, ad2, zm2, zs2)

    s2pp = s2[0].transpose(1, 0)

    out = pl.pallas_call(
        _tc_c,
        out_shape=jax.ShapeDtypeStruct((NPAD, D_OUT), f32),
    )(msg2[0], msg2[1], s2pp, ones16, b2.reshape(1, -1))
    return out[:N]


# speedup vs baseline: 31.5406x; 1.0021x over previous
import functools

import jax
import jax.numpy as jnp
from jax import lax
from jax.experimental import pallas as pl
from jax.experimental.pallas import tpu as pltpu
from jax.experimental.pallas import tpu_sc as plsc

N = 10000
D_IN = 128
HID = 8
HEADS = 8
D_OUT = 128

NPAD = 10240
NC, NS = 2, 16
NW = NC * NS
RPT = NPAD // NS
B = 128
PBLK = 42
NBLK1 = 2 * PBLK
NBLK2 = 4 * PBLK

W1ROW = 80
A1ROW = 16


def _tc_a(x_ref, w_ref, wa_ref, hs_ref, ad_ref):
    x = x_ref[...]
    hs_ref[...] = jnp.dot(x, w_ref[...], preferred_element_type=jnp.float32)
    ad_ref[...] = jnp.dot(x, wa_ref[...], preferred_element_type=jnp.float32)


def _tc_b(m0_ref, m1_ref, s0_ref, s1_ref, r8_ref, w2_ref, waa_ref, b1_ref,
          hs2a_ref, hs2b_ref, aa_ref):
    srep = jnp.dot(s0_ref[...] + s1_ref[...], r8_ref[...],
                   preferred_element_type=jnp.float32)
    g = (m0_ref[...] + m1_ref[...]) / (srep + 1e-16) + b1_ref[...]
    g = jnp.where(g > 0, g, jnp.exp(g) - 1.0)
    h2 = jnp.dot(g, w2_ref[...], preferred_element_type=jnp.float32)
    hs2a_ref[...] = h2[:, 0:64]
    hs2b_ref[...] = h2[:, 64:128]
    aa_ref[...] = jnp.dot(g, waa_ref[...], preferred_element_type=jnp.float32)


def _tc_c(m0_ref, m1_ref, sp_ref, ones_ref, b2_ref, out_ref):
    s = jnp.dot(sp_ref[...], ones_ref[...],
                preferred_element_type=jnp.float32)
    v = (jnp.concatenate([m0_ref[...], m1_ref[...]], axis=1) / (s + 1e-16)
         + b2_ref[...])
    m = jnp.max(v, axis=1, keepdims=True)
    u = v - m
    lse = jnp.log(jnp.sum(jnp.exp(u), axis=1, keepdims=True))
    out_ref[...] = u - lse


def _sc_edge1(src_h, dst_h, hs_h, ad_h, zm_h, zs_h, msg_o, s_o,
              sidxa, didxa, hsr0, hsr1, adr0, adr1, outr0, outr1, exr0, exr1,
              exbuf, accm, accs, gs0, gs1):
    cid = lax.axis_index("c")
    sid = lax.axis_index("s")
    wid = sid * NC + cid
    pltpu.sync_copy(zm_h.at[pl.ds(sid * RPT, RPT)],
                    accm.at[pl.ds(sid * RPT, RPT)])
    pltpu.sync_copy(zs_h.at[pl.ds(sid * RPT, RPT)],
                    accs.at[pl.ds(sid * RPT, RPT)])
    plsc.subcore_barrier()
    lane = lax.broadcasted_iota(jnp.int32, (16,), 0)
    half = lane // 8
    zero16 = lane // 16
    cols = [half + 2 * k for k in range(4)]
    lane7 = jnp.bitwise_and(lane, 7)
    m8 = lane < 8

    def gathers(i, hsr, adr, gsem):
        a = pltpu.async_copy(hs_h.at[sidxa.at[i]], hsr, gsem)
        b = pltpu.async_copy(ad_h.at[didxa.at[i]], adr, gsem)
        return a, b

    def drain(i, hsr, adr, gsem):
        pltpu.make_async_copy(hs_h.at[sidxa.at[i]], hsr, gsem).wait()
        pltpu.make_async_copy(ad_h.at[didxa.at[i]], adr, gsem).wait()

    def compute(hsr, adr, outr, exr):
        def edge(e, c2):
            asv = hsr[e, pl.ds(64, 16)]
            adv = adr[e, :]
            ev = asv + adv
            ev = jnp.maximum(ev, 0.2 * ev)
            ex = jnp.exp(ev)
            exbuf[...] = ex
            rowi = zero16 + e
            plsc.store_scatter(exr, [rowi, lane7], ex, mask=m8)
            for k in range(4):
                exb = plsc.load_gather(exbuf, [cols[k]])
                outr[e, pl.ds(16 * k, 16)] = hsr[e, pl.ds(16 * k, 16)] * exb
            return c2

        lax.fori_loop(0, B, edge, 0, unroll=2)

    def scatter(i, outr, exr):
        pltpu.sync_copy(outr, accm.at[didxa.at[i]], add=True)
        pltpu.sync_copy(exr, accs.at[didxa.at[i]], add=True)

    for p in range(NBLK1 // PBLK):
        row0 = wid * NBLK1 + p * PBLK
        pltpu.sync_copy(src_h.at[pl.ds(row0, PBLK)], sidxa)
        pltpu.sync_copy(dst_h.at[pl.ds(row0, PBLK)], didxa)
        gathers(0, hsr0, adr0, gs0)

        def body(j, carry):
            i0 = 2 * j
            i1 = i0 + 1
            c1a, c1b = gathers(i1, hsr1, adr1, gs1)
            drain(i0, hsr0, adr0, gs0)
            compute(hsr0, adr0, outr0, exr0)

            @pl.when(i0 + 2 < PBLK)
            def _():
                gathers(i0 + 2, hsr0, adr0, gs0)

            scatter(i0, outr0, exr0)
            c1a.wait()
            c1b.wait()
            compute(hsr1, adr1, outr1, exr1)

            @pl.when(i1 + 2 < PBLK)
            def _():
                gathers(i1 + 2, hsr1, adr1, gs1)

            scatter(i1, outr1, exr1)
            return carry

        lax.fori_loop(0, PBLK // 2, body, 0)

    plsc.subcore_barrier()
    pltpu.sync_copy(accm.at[pl.ds(sid * RPT, RPT)],
                    msg_o.at[cid].at[pl.ds(sid * RPT, RPT)])
    pltpu.sync_copy(accs.at[pl.ds(sid * RPT, RPT)],
                    s_o.at[cid].at[pl.ds(sid * RPT, RPT)])


def _sc_edge2(src_h, dst_h, hs2a_h, hs2b_h, as2_h, ad2_h, zm_h, zs_h,
              msg_o, s_o,
              sidxa, didxa, h2r0, h2r1, outr0, outr1, astab, adtab, exbuf,
              s2loc, accm, gs0, gs1):
    cid = lax.axis_index("c")
    sid = lax.axis_index("s")
    pltpu.sync_copy(as2_h, astab)
    pltpu.sync_copy(ad2_h, adtab)
    pltpu.sync_copy(zs_h, s2loc)
    pltpu.sync_copy(zm_h.at[pl.ds(sid * RPT, RPT)],
                    accm.at[pl.ds(sid * RPT, RPT)])
    plsc.subcore_barrier()
    lane = lax.broadcasted_iota(jnp.int32, (16,), 0)
    zero16 = lane // 16
    jcols = [zero16 + j for j in range(16)]
    m1 = lane < 1

    def gathers(i, h2r, gsem):
        @pl.when(cid == 0)
        def _():
            pltpu.async_copy(hs2a_h.at[sidxa.at[i]], h2r, gsem)

        @pl.when(cid == 1)
        def _():
            pltpu.async_copy(hs2b_h.at[sidxa.at[i]], h2r, gsem)

    def drain(i, h2r, gsem):
        pltpu.make_async_copy(hs2a_h.at[sidxa.at[i]], h2r, gsem).wait()

    def scatter(i, outr):
        pltpu.sync_copy(outr, accm.at[didxa.at[i]], add=True)

    for p in range(NBLK2 // PBLK):
        row0 = sid * NBLK2 + p * PBLK
        pltpu.sync_copy(src_h.at[pl.ds(row0, PBLK)], sidxa)
        pltpu.sync_copy(dst_h.at[pl.ds(row0, PBLK)], didxa)
        gathers(0, h2r0, gs0)

        def body(j, carry):
            i0 = 2 * j
            i1 = i0 + 1
            gathers(i1, h2r1, gs1)
            drain(i0, h2r0, gs0)
            _compute2(i0, sidxa, didxa, h2r0, outr0, astab, adtab, exbuf,
                      s2loc, zero16, jcols, m1)

            @pl.when(i0 + 2 < PBLK)
            def _():
                gathers(i0 + 2, h2r0, gs0)

            scatter(i0, outr0)
            drain(i1, h2r1, gs1)
            _compute2(i1, sidxa, didxa, h2r1, outr1, astab, adtab, exbuf,
                      s2loc, zero16, jcols, m1)

            @pl.when(i1 + 2 < PBLK)
            def _():
                gathers(i1 + 2, h2r1, gs1)

            scatter(i1, outr1)
            return carry

        lax.fori_loop(0, PBLK // 2, body, 0)

    plsc.subcore_barrier()
    pltpu.sync_copy(accm.at[pl.ds(sid * RPT, RPT)],
                    msg_o.at[cid].at[pl.ds(sid * RPT, RPT)])
    pltpu.sync_copy(s2loc, s_o.at[cid].at[sid])


def _compute2(i, sidxa, didxa, h2r, outr, astab, adtab, exbuf, s2loc,
              zero16, jcols, m1):
    def grp(g, c2):
        s16 = sidxa[i, pl.ds(g * 16, 16)]
        d16 = didxa[i, pl.ds(g * 16, 16)]
        asg = plsc.load_gather(astab, [s16])
        adg = plsc.load_gather(adtab, [d16])
        ev = asg + adg
        ev = jnp.maximum(ev, 0.2 * ev)
        ex16 = jnp.exp(ev)
        exbuf[...] = ex16
        for j in range(16):
            e = g * 16 + j
            exb = plsc.load_gather(exbuf, [jcols[j]])
            dv = plsc.load_gather(didxa, [zero16 + i, zero16 + e])
            plsc.addupdate_scatter(s2loc, [dv], exb, mask=m1)
            for k in range(4):
                outr[e, pl.ds(16 * k, 16)] = h2r[e, pl.ds(16 * k, 16)] * exb
        return c2

    lax.fori_loop(0, B // 16, grp, 0)


def kernel(x, edge_index, W1, a_src1, a_dst1, b1, W2, a_src2, a_dst2, b2):
    f32 = jnp.float32
    i32 = jnp.int32

    e_real = edge_index.shape[1] + N
    e_pad = NW * NBLK1 * B
    assert e_pad >= e_real and NW * NBLK1 == NS * NBLK2
    loop = jnp.arange(N, dtype=i32)
    pad = jnp.full((e_pad - e_real,), N, dtype=i32)
    src = jnp.concatenate([edge_index[0].astype(i32), loop, pad])
    dst = jnp.concatenate([edge_index[1].astype(i32), loop, pad])
    src2d = src.reshape(NW * NBLK1, B)
    dst2d = dst.reshape(NW * NBLK1, B)

    k64 = jnp.arange(HEADS * HID)
    as64 = jnp.zeros((HEADS * HID, HEADS), f32).at[k64, k64 // HID].set(
        a_src1.reshape(-1))
    ad64 = jnp.zeros((HEADS * HID, HEADS), f32).at[k64, k64 // HID].set(
        a_dst1.reshape(-1))
    r8 = jnp.zeros((HEADS, HEADS * HID), f32).at[k64 // HID, k64].set(1.0)
    w1cat = jnp.concatenate(
        [W1, W1 @ as64, jnp.zeros((D_IN, 8), f32)], axis=1)
    w1ad = jnp.concatenate(
        [W1 @ ad64, jnp.zeros((D_IN, 8), f32)], axis=1)
    waa = jnp.concatenate(
        [W2 @ a_src2.reshape(-1, 1), W2 @ a_dst2.reshape(-1, 1),
         jnp.zeros((HEADS * HID, 14), f32)], axis=1)
    ones16 = jnp.ones((NS, 1), f32)

    x_pad = jnp.zeros((NPAD, D_IN), f32).at[:N].set(x)

    hs1, ad1 = pl.pallas_call(
        _tc_a,
        out_shape=(jax.ShapeDtypeStruct((NPAD, W1ROW), f32),
                   jax.ShapeDtypeStruct((NPAD, A1ROW), f32)),
    )(x_pad, w1cat, w1ad)

    zm1 = jnp.zeros((NPAD, 64), f32)
    zs1 = jnp.zeros((NPAD, 8), f32)
    mesh = plsc.VectorSubcoreMesh(core_axis_name="c", subcore_axis_name="s")
    sc_params = pltpu.CompilerParams(
        use_tc_tiling_on_sc=False, needs_layout_passes=False)
    msg1, s1 = pl.kernel(
        _sc_edge1,
        out_type=(jax.ShapeDtypeStruct((NC, NPAD, 64), f32),
                  jax.ShapeDtypeStruct((NC, NPAD, 8), f32)),
        mesh=mesh,
        compiler_params=sc_params,
        scratch_types=[
            pltpu.VMEM((PBLK, B), i32),
            pltpu.VMEM((PBLK, B), i32),
            pltpu.VMEM((B, W1ROW), f32),
            pltpu.VMEM((B, W1ROW), f32),
            pltpu.VMEM((B, A1ROW), f32),
            pltpu.VMEM((B, A1ROW), f32),
            pltpu.VMEM((B, 64), f32),
            pltpu.VMEM((B, 64), f32),
            pltpu.VMEM((B, 8), f32),
            pltpu.VMEM((B, 8), f32),
            pltpu.VMEM((16,), f32),
            pltpu.VMEM_SHARED((NPAD, 64), f32),
            pltpu.VMEM_SHARED((NPAD, 8), f32),
            pltpu.SemaphoreType.DMA,
            pltpu.SemaphoreType.DMA,
        ],
    )(src2d, dst2d, hs1, ad1, zm1, zs1)

    hs2a, hs2b, aa = pl.pallas_call(
        _tc_b,
        out_shape=(jax.ShapeDtypeStruct((NPAD, 64), f32),
                   jax.ShapeDtypeStruct((NPAD, 64), f32),
                   jax.ShapeDtypeStruct((NPAD, 16), f32)),
    )(msg1[0], msg1[1], s1[0], s1[1], r8, W2, waa, b1.reshape(1, -1))

    as2 = aa[:, 0].reshape(NPAD)
    ad2 = aa[:, 1].reshape(NPAD)

    zm2 = jnp.zeros((NPAD, 64), f32)
    zs2 = jnp.zeros((NPAD,), f32)
    msg2, s2 = pl.kernel(
        _sc_edge2,
        out_type=(jax.ShapeDtypeStruct((NC, NPAD, 64), f32),
                  jax.ShapeDtypeStruct((NC, NS, NPAD), f32)),
        mesh=mesh,
        compiler_params=sc_params,
        scratch_types=[
            pltpu.VMEM((PBLK, B), i32),
            pltpu.VMEM((PBLK, B), i32),
            pltpu.VMEM((B, 64), f32),
            pltpu.VMEM((B, 64), f32),
            pltpu.VMEM((B, 64), f32),
            pltpu.VMEM((B, 64), f32),
            pltpu.VMEM((NPAD,), f32),
            pltpu.VMEM((NPAD,), f32),
            pltpu.VMEM((16,), f32),
            pltpu.VMEM((NPAD,), f32),
            pltpu.VMEM_SHARED((NPAD, 64), f32),
            pltpu.SemaphoreType.DMA,
            pltpu.SemaphoreType.DMA,
        ],
    )(src2d, dst2d, hs2a, hs2b, as2, ad2, zm2, zs2)

    s2pp = s2[0].transpose(1, 0)

    out = pl.pallas_call(
        _tc_c,
        out_shape=jax.ShapeDtypeStruct((NPAD, D_OUT), f32),
    )(msg2[0], msg2[1], s2pp, ones16, b2.reshape(1, -1))
    return out[:N]


# speedup vs baseline: 32.7160x; 1.0373x over previous
import functools

import jax
import jax.numpy as jnp
from jax import lax
from jax.experimental import pallas as pl
from jax.experimental.pallas import tpu as pltpu
from jax.experimental.pallas import tpu_sc as plsc

N = 10000
D_IN = 128
HID = 8
HEADS = 8
D_OUT = 128

NPAD = 10240
NC, NS = 2, 16
NW = NC * NS
RPT = NPAD // NS
B = 128
PBLK = 42
NBLK1 = 2 * PBLK
NBLK2 = 4 * PBLK

W1ROW = 80
A1ROW = 16


def _tc_a(x_ref, w_ref, wa_ref, hs_ref, ad_ref):
    x = x_ref[...]
    hs_ref[...] = jnp.dot(x, w_ref[...], preferred_element_type=jnp.float32)
    ad_ref[...] = jnp.dot(x, wa_ref[...], preferred_element_type=jnp.float32)


def _tc_b(m0_ref, m1_ref, r8_ref, w2_ref, waa_ref, b1_ref,
          hs2a_ref, hs2b_ref, aa_ref):
    acc = m0_ref[...] + m1_ref[...]
    srep = jnp.dot(acc[:, 64:72], r8_ref[...],
                   preferred_element_type=jnp.float32)
    g = acc[:, 0:64] / (srep + 1e-16) + b1_ref[...]
    g = jnp.where(g > 0, g, jnp.exp(g) - 1.0)
    h2 = jnp.dot(g, w2_ref[...], preferred_element_type=jnp.float32)
    hs2a_ref[...] = h2[:, 0:64]
    hs2b_ref[...] = h2[:, 64:128]
    aa_ref[...] = jnp.dot(g, waa_ref[...], preferred_element_type=jnp.float32)


def _tc_c(m0_ref, m1_ref, sp_ref, ones_ref, b2_ref, out_ref):
    s = jnp.dot(sp_ref[...], ones_ref[...],
                preferred_element_type=jnp.float32)
    v = (jnp.concatenate([m0_ref[...], m1_ref[...]], axis=1) / (s + 1e-16)
         + b2_ref[...])
    m = jnp.max(v, axis=1, keepdims=True)
    u = v - m
    lse = jnp.log(jnp.sum(jnp.exp(u), axis=1, keepdims=True))
    out_ref[...] = u - lse


def _sc_edge1(src_h, dst_h, hs_h, ad_h, zm_h, msg_o,
              sidxa, didxa, hsr0, hsr1, adr0, adr1, outr0, outr1,
              exbuf, accm, gs0, gs1):
    cid = lax.axis_index("c")
    sid = lax.axis_index("s")
    wid = sid * NC + cid
    pltpu.sync_copy(zm_h.at[pl.ds(sid * RPT, RPT)],
                    accm.at[pl.ds(sid * RPT, RPT)])
    plsc.subcore_barrier()
    lane = lax.broadcasted_iota(jnp.int32, (16,), 0)
    half = lane // 8
    zero16 = lane // 16
    cols = [half + 2 * k for k in range(4)]
    lane7 = jnp.bitwise_and(lane, 7)
    m8 = lane < 8

    def gathers(i, hsr, adr, gsem):
        a = pltpu.async_copy(hs_h.at[sidxa.at[i]], hsr, gsem)
        b = pltpu.async_copy(ad_h.at[didxa.at[i]], adr, gsem)
        return a, b

    def drain(i, hsr, adr, gsem):
        pltpu.make_async_copy(hs_h.at[sidxa.at[i]], hsr, gsem).wait()
        pltpu.make_async_copy(ad_h.at[didxa.at[i]], adr, gsem).wait()

    def compute(hsr, adr, outr):
        def edge(e, c2):
            asv = hsr[e, pl.ds(64, 16)]
            adv = adr[e, :]
            ev = asv + adv
            ev = jnp.maximum(ev, 0.2 * ev)
            ex = jnp.exp(ev)
            exbuf[...] = ex
            rowi = zero16 + e
            plsc.store_scatter(outr, [rowi, 64 + lane7], ex, mask=m8)
            for k in range(4):
                exb = plsc.load_gather(exbuf, [cols[k]])
                outr[e, pl.ds(16 * k, 16)] = hsr[e, pl.ds(16 * k, 16)] * exb
            return c2

        lax.fori_loop(0, B, edge, 0, unroll=2)

    def scatter(i, outr):
        pltpu.sync_copy(outr, accm.at[didxa.at[i]], add=True)

    for p in range(NBLK1 // PBLK):
        row0 = wid * NBLK1 + p * PBLK
        pltpu.sync_copy(src_h.at[pl.ds(row0, PBLK)], sidxa)
        pltpu.sync_copy(dst_h.at[pl.ds(row0, PBLK)], didxa)
        gathers(0, hsr0, adr0, gs0)

        def body(j, carry):
            i0 = 2 * j
            i1 = i0 + 1
            c1a, c1b = gathers(i1, hsr1, adr1, gs1)
            drain(i0, hsr0, adr0, gs0)
            compute(hsr0, adr0, outr0)

            @pl.when(i0 + 2 < PBLK)
            def _():
                gathers(i0 + 2, hsr0, adr0, gs0)

            scatter(i0, outr0)
            c1a.wait()
            c1b.wait()
            compute(hsr1, adr1, outr1)

            @pl.when(i1 + 2 < PBLK)
            def _():
                gathers(i1 + 2, hsr1, adr1, gs1)

            scatter(i1, outr1)
            return carry

        lax.fori_loop(0, PBLK // 2, body, 0)

    plsc.subcore_barrier()
    pltpu.sync_copy(accm.at[pl.ds(sid * RPT, RPT)],
                    msg_o.at[cid].at[pl.ds(sid * RPT, RPT)])


def _sc_edge2(src_h, dst_h, hs2a_h, hs2b_h, as2_h, ad2_h, zm_h, zs_h,
              msg_o, s_o,
              sidxa, didxa, h2r0, h2r1, outr0, outr1, astab, adtab, exbuf,
              s2loc, accm, gs0, gs1):
    cid = lax.axis_index("c")
    sid = lax.axis_index("s")
    pltpu.sync_copy(as2_h, astab)
    pltpu.sync_copy(ad2_h, adtab)
    pltpu.sync_copy(zs_h, s2loc)
    pltpu.sync_copy(zm_h.at[pl.ds(sid * RPT, RPT)],
                    accm.at[pl.ds(sid * RPT, RPT)])
    plsc.subcore_barrier()
    lane = lax.broadcasted_iota(jnp.int32, (16,), 0)
    zero16 = lane // 16
    jcols = [zero16 + j for j in range(16)]
    m1 = lane < 1

    def gathers(i, h2r, gsem):
        @pl.when(cid == 0)
        def _():
            pltpu.async_copy(hs2a_h.at[sidxa.at[i]], h2r, gsem)

        @pl.when(cid == 1)
        def _():
            pltpu.async_copy(hs2b_h.at[sidxa.at[i]], h2r, gsem)

    def drain(i, h2r, gsem):
        pltpu.make_async_copy(hs2a_h.at[sidxa.at[i]], h2r, gsem).wait()

    def scatter(i, outr):
        pltpu.sync_copy(outr, accm.at[didxa.at[i]], add=True)

    for p in range(NBLK2 // PBLK):
        row0 = sid * NBLK2 + p * PBLK
        pltpu.sync_copy(src_h.at[pl.ds(row0, PBLK)], sidxa)
        pltpu.sync_copy(dst_h.at[pl.ds(row0, PBLK)], didxa)
        gathers(0, h2r0, gs0)

        def body(j, carry):
            i0 = 2 * j
            i1 = i0 + 1
            gathers(i1, h2r1, gs1)
            drain(i0, h2r0, gs0)
            _compute2(i0, sidxa, didxa, h2r0, outr0, astab, adtab, exbuf,
                      s2loc, zero16, jcols, m1)

            @pl.when(i0 + 2 < PBLK)
            def _():
                gathers(i0 + 2, h2r0, gs0)

            scatter(i0, outr0)
            drain(i1, h2r1, gs1)
            _compute2(i1, sidxa, didxa, h2r1, outr1, astab, adtab, exbuf,
                      s2loc, zero16, jcols, m1)

            @pl.when(i1 + 2 < PBLK)
            def _():
                gathers(i1 + 2, h2r1, gs1)

            scatter(i1, outr1)
            return carry

        lax.fori_loop(0, PBLK // 2, body, 0)

    plsc.subcore_barrier()
    pltpu.sync_copy(accm.at[pl.ds(sid * RPT, RPT)],
                    msg_o.at[cid].at[pl.ds(sid * RPT, RPT)])
    pltpu.sync_copy(s2loc, s_o.at[cid].at[sid])


def _compute2(i, sidxa, didxa, h2r, outr, astab, adtab, exbuf, s2loc,
              zero16, jcols, m1):
    def grp(g, c2):
        s16 = sidxa[i, pl.ds(g * 16, 16)]
        d16 = didxa[i, pl.ds(g * 16, 16)]
        asg = plsc.load_gather(astab, [s16])
        adg = plsc.load_gather(adtab, [d16])
        ev = asg + adg
        ev = jnp.maximum(ev, 0.2 * ev)
        ex16 = jnp.exp(ev)
        exbuf[...] = ex16
        for j in range(16):
            e = g * 16 + j
            exb = plsc.load_gather(exbuf, [jcols[j]])
            dv = plsc.load_gather(didxa, [zero16 + i, zero16 + e])
            plsc.addupdate_scatter(s2loc, [dv], exb, mask=m1)
            for k in range(4):
                outr[e, pl.ds(16 * k, 16)] = h2r[e, pl.ds(16 * k, 16)] * exb
        return c2

    lax.fori_loop(0, B // 16, grp, 0)


def kernel(x, edge_index, W1, a_src1, a_dst1, b1, W2, a_src2, a_dst2, b2):
    f32 = jnp.float32
    i32 = jnp.int32

    e_real = edge_index.shape[1] + N
    e_pad = NW * NBLK1 * B
    assert e_pad >= e_real and NW * NBLK1 == NS * NBLK2
    loop = jnp.arange(N, dtype=i32)
    pad = jnp.full((e_pad - e_real,), N, dtype=i32)
    src = jnp.concatenate([edge_index[0].astype(i32), loop, pad])
    dst = jnp.concatenate([edge_index[1].astype(i32), loop, pad])
    src2d = src.reshape(NW * NBLK1, B)
    dst2d = dst.reshape(NW * NBLK1, B)

    k64 = jnp.arange(HEADS * HID)
    as64 = jnp.zeros((HEADS * HID, HEADS), f32).at[k64, k64 // HID].set(
        a_src1.reshape(-1))
    ad64 = jnp.zeros((HEADS * HID, HEADS), f32).at[k64, k64 // HID].set(
        a_dst1.reshape(-1))
    r8 = jnp.zeros((HEADS, HEADS * HID), f32).at[k64 // HID, k64].set(1.0)
    w1cat = jnp.concatenate(
        [W1, W1 @ as64, jnp.zeros((D_IN, 8), f32)], axis=1)
    w1ad = jnp.concatenate(
        [W1 @ ad64, jnp.zeros((D_IN, 8), f32)], axis=1)
    waa = jnp.concatenate(
        [W2 @ a_src2.reshape(-1, 1), W2 @ a_dst2.reshape(-1, 1),
         jnp.zeros((HEADS * HID, 14), f32)], axis=1)
    ones16 = jnp.ones((NS, 1), f32)

    x_pad = jnp.zeros((NPAD, D_IN), f32).at[:N].set(x)

    hs1, ad1 = pl.pallas_call(
        _tc_a,
        out_shape=(jax.ShapeDtypeStruct((NPAD, W1ROW), f32),
                   jax.ShapeDtypeStruct((NPAD, A1ROW), f32)),
    )(x_pad, w1cat, w1ad)

    zm1 = jnp.zeros((NPAD, 72), f32)
    mesh = plsc.VectorSubcoreMesh(core_axis_name="c", subcore_axis_name="s")
    sc_params = pltpu.CompilerParams(
        use_tc_tiling_on_sc=False, needs_layout_passes=False)
    msg1 = pl.kernel(
        _sc_edge1,
        out_type=jax.ShapeDtypeStruct((NC, NPAD, 72), f32),
        mesh=mesh,
        compiler_params=sc_params,
        scratch_types=[
            pltpu.VMEM((PBLK, B), i32),
            pltpu.VMEM((PBLK, B), i32),
            pltpu.VMEM((B, W1ROW), f32),
            pltpu.VMEM((B, W1ROW), f32),
            pltpu.VMEM((B, A1ROW), f32),
            pltpu.VMEM((B, A1ROW), f32),
            pltpu.VMEM((B, 72), f32),
            pltpu.VMEM((B, 72), f32),
            pltpu.VMEM((16,), f32),
            pltpu.VMEM_SHARED((NPAD, 72), f32),
            pltpu.SemaphoreType.DMA,
            pltpu.SemaphoreType.DMA,
        ],
    )(src2d, dst2d, hs1, ad1, zm1)

    hs2a, hs2b, aa = pl.pallas_call(
        _tc_b,
        out_shape=(jax.ShapeDtypeStruct((NPAD, 64), f32),
                   jax.ShapeDtypeStruct((NPAD, 64), f32),
                   jax.ShapeDtypeStruct((NPAD, 16), f32)),
    )(msg1[0], msg1[1], r8, W2, waa, b1.reshape(1, -1))

    as2 = aa[:, 0].reshape(NPAD)
    ad2 = aa[:, 1].reshape(NPAD)

    zm2 = jnp.zeros((NPAD, 64), f32)
    zs2 = jnp.zeros((NPAD,), f32)
    msg2, s2 = pl.kernel(
        _sc_edge2,
        out_type=(jax.ShapeDtypeStruct((NC, NPAD, 64), f32),
                  jax.ShapeDtypeStruct((NC, NS, NPAD), f32)),
        mesh=mesh,
        compiler_params=sc_params,
        scratch_types=[
            pltpu.VMEM((PBLK, B), i32),
            pltpu.VMEM((PBLK, B), i32),
            pltpu.VMEM((B, 64), f32),
            pltpu.VMEM((B, 64), f32),
            pltpu.VMEM((B, 64), f32),
            pltpu.VMEM((B, 64), f32),
            pltpu.VMEM((NPAD,), f32),
            pltpu.VMEM((NPAD,), f32),
            pltpu.VMEM((16,), f32),
            pltpu.VMEM((NPAD,), f32),
            pltpu.VMEM_SHARED((NPAD, 64), f32),
            pltpu.SemaphoreType.DMA,
            pltpu.SemaphoreType.DMA,
        ],
    )(src2d, dst2d, hs2a, hs2b, as2, ad2, zm2, zs2)

    s2pp = s2[0].transpose(1, 0)

    out = pl.pallas_call(
        _tc_c,
        out_shape=jax.ShapeDtypeStruct((NPAD, D_OUT), f32),
    )(msg2[0], msg2[1], s2pp, ones16, b2.reshape(1, -1))
    return out[:N]


# speedup vs baseline: 38.9515x; 1.1906x over previous
import functools

import jax
import jax.numpy as jnp
from jax import lax
from jax.experimental import pallas as pl
from jax.experimental.pallas import tpu as pltpu
from jax.experimental.pallas import tpu_sc as plsc

N = 10000
D_IN = 128
HID = 8
HEADS = 8
D_OUT = 128

NPAD = 10240
NC, NS = 2, 16
NW = NC * NS
RPT = NPAD // NS
B = 128
PBLK = 42
NBLK1 = 2 * PBLK
NBLK2 = 4 * PBLK

W1ROW = 80
A1ROW = 16


def _tc_a(x_ref, w_ref, wa_ref, hs_ref, ad_ref):
    x = x_ref[...]
    hs_ref[...] = jnp.dot(x, w_ref[...], preferred_element_type=jnp.float32)
    ad_ref[...] = jnp.dot(x, wa_ref[...], preferred_element_type=jnp.float32)


def _tc_b(m0_ref, m1_ref, r8_ref, w2a_ref, w2b_ref, waa_ref, b1_ref,
          hs2a_ref, hs2b_ref, aa_ref):
    acc = m0_ref[...] + m1_ref[...]
    srep = jnp.dot(acc[:, 64:72], r8_ref[...],
                   preferred_element_type=jnp.float32)
    g = acc[:, 0:64] / (srep + 1e-16) + b1_ref[...]
    g = jnp.where(g > 0, g, jnp.exp(g) - 1.0)
    hs2a_ref[...] = jnp.dot(
        g, w2a_ref[...], preferred_element_type=jnp.float32
    ).astype(jnp.bfloat16)
    hs2b_ref[...] = jnp.dot(
        g, w2b_ref[...], preferred_element_type=jnp.float32
    ).astype(jnp.bfloat16)
    aa_ref[...] = jnp.dot(g, waa_ref[...], preferred_element_type=jnp.float32)


def _tc_c(m0_ref, m1_ref, sp_ref, ones_ref, b2_ref, out_ref):
    s = jnp.dot(sp_ref[...], ones_ref[...],
                preferred_element_type=jnp.float32)
    v = (jnp.concatenate([m0_ref[...], m1_ref[...]], axis=1) / (s + 1e-16)
         + b2_ref[...])
    m = jnp.max(v, axis=1, keepdims=True)
    u = v - m
    lse = jnp.log(jnp.sum(jnp.exp(u), axis=1, keepdims=True))
    out_ref[...] = u - lse


def _sc_edge1(src_h, dst_h, hs_h, ad_h, zm_h, msg_o,
              sidxa, didxa, hsr0, hsr1, adr0, adr1, outr0, outr1,
              exbuf, accm, gs0, gs1):
    cid = lax.axis_index("c")
    sid = lax.axis_index("s")
    wid = sid * NC + cid
    pltpu.sync_copy(zm_h.at[pl.ds(sid * RPT, RPT)],
                    accm.at[pl.ds(sid * RPT, RPT)])
    plsc.subcore_barrier()
    lane = lax.broadcasted_iota(jnp.int32, (16,), 0)
    half = lane // 8
    zero16 = lane // 16
    cols = [half + 2 * k for k in range(4)]
    lane7 = jnp.bitwise_and(lane, 7)
    m8 = lane < 8

    def gathers(i, hsr, adr, gsem):
        a = pltpu.async_copy(hs_h.at[sidxa.at[i]], hsr, gsem)
        b = pltpu.async_copy(ad_h.at[didxa.at[i]], adr, gsem)
        return a, b

    def drain(i, hsr, adr, gsem):
        pltpu.make_async_copy(hs_h.at[sidxa.at[i]], hsr, gsem).wait()
        pltpu.make_async_copy(ad_h.at[didxa.at[i]], adr, gsem).wait()

    def compute(hsr, adr, outr):
        def edge(e, c2):
            asv = hsr[e, pl.ds(64, 16)]
            adv = adr[e, :]
            ev = asv + adv
            ev = jnp.maximum(ev, 0.2 * ev)
            ex = jnp.exp(ev)
            exbuf[...] = ex
            rowi = zero16 + e
            plsc.store_scatter(outr, [rowi, 64 + lane7], ex, mask=m8)
            for k in range(4):
                exb = plsc.load_gather(exbuf, [cols[k]])
                outr[e, pl.ds(16 * k, 16)] = hsr[e, pl.ds(16 * k, 16)] * exb
            return c2

        lax.fori_loop(0, B, edge, 0, unroll=2)

    def scatter(i, outr):
        pltpu.sync_copy(outr, accm.at[didxa.at[i]], add=True)

    for p in range(NBLK1 // PBLK):
        row0 = wid * NBLK1 + p * PBLK
        pltpu.sync_copy(src_h.at[pl.ds(row0, PBLK)], sidxa)
        pltpu.sync_copy(dst_h.at[pl.ds(row0, PBLK)], didxa)
        gathers(0, hsr0, adr0, gs0)

        def body(j, carry):
            i0 = 2 * j
            i1 = i0 + 1
            c1a, c1b = gathers(i1, hsr1, adr1, gs1)
            drain(i0, hsr0, adr0, gs0)
            compute(hsr0, adr0, outr0)

            @pl.when(i0 + 2 < PBLK)
            def _():
                gathers(i0 + 2, hsr0, adr0, gs0)

            scatter(i0, outr0)
            c1a.wait()
            c1b.wait()
            compute(hsr1, adr1, outr1)

            @pl.when(i1 + 2 < PBLK)
            def _():
                gathers(i1 + 2, hsr1, adr1, gs1)

            scatter(i1, outr1)
            return carry

        lax.fori_loop(0, PBLK // 2, body, 0)

    plsc.subcore_barrier()
    pltpu.sync_copy(accm.at[pl.ds(sid * RPT, RPT)],
                    msg_o.at[cid].at[pl.ds(sid * RPT, RPT)])


def _sc_edge2(src_h, dst_h, hs2a_h, hs2b_h, as2_h, ad2_h, zm_h, zs_h,
              msg_o, s_o,
              sidxa, didxa, h2r0, h2r1, outr0, outr1, astab, adtab, exbuf,
              s2loc, accm, gs0, gs1):
    cid = lax.axis_index("c")
    sid = lax.axis_index("s")
    pltpu.sync_copy(as2_h, astab)
    pltpu.sync_copy(ad2_h, adtab)
    pltpu.sync_copy(zs_h, s2loc)
    pltpu.sync_copy(zm_h.at[pl.ds(sid * RPT, RPT)],
                    accm.at[pl.ds(sid * RPT, RPT)])
    plsc.subcore_barrier()
    lane = lax.broadcasted_iota(jnp.int32, (16,), 0)
    zero16 = lane // 16
    jcols = [zero16 + j for j in range(16)]
    m1 = lane < 1

    def gathers(i, h2r, gsem):
        @pl.when(cid == 0)
        def _():
            pltpu.async_copy(hs2a_h.at[sidxa.at[i]], h2r, gsem)

        @pl.when(cid == 1)
        def _():
            pltpu.async_copy(hs2b_h.at[sidxa.at[i]], h2r, gsem)

    def drain(i, h2r, gsem):
        pltpu.make_async_copy(hs2a_h.at[sidxa.at[i]], h2r, gsem).wait()

    def scatter(i, outr):
        pltpu.sync_copy(outr, accm.at[didxa.at[i]], add=True)

    for p in range(NBLK2 // PBLK):
        row0 = sid * NBLK2 + p * PBLK
        pltpu.sync_copy(src_h.at[pl.ds(row0, PBLK)], sidxa)
        pltpu.sync_copy(dst_h.at[pl.ds(row0, PBLK)], didxa)
        gathers(0, h2r0, gs0)

        def body(j, carry):
            i0 = 2 * j
            i1 = i0 + 1
            gathers(i1, h2r1, gs1)
            drain(i0, h2r0, gs0)
            _compute2(i0, sidxa, didxa, h2r0, outr0, astab, adtab, exbuf,
                      s2loc, zero16, jcols, m1)

            @pl.when(i0 + 2 < PBLK)
            def _():
                gathers(i0 + 2, h2r0, gs0)

            scatter(i0, outr0)
            drain(i1, h2r1, gs1)
            _compute2(i1, sidxa, didxa, h2r1, outr1, astab, adtab, exbuf,
                      s2loc, zero16, jcols, m1)

            @pl.when(i1 + 2 < PBLK)
            def _():
                gathers(i1 + 2, h2r1, gs1)

            scatter(i1, outr1)
            return carry

        lax.fori_loop(0, PBLK // 2, body, 0)

    plsc.subcore_barrier()
    pltpu.sync_copy(accm.at[pl.ds(sid * RPT, RPT)],
                    msg_o.at[cid].at[pl.ds(sid * RPT, RPT)])
    pltpu.sync_copy(s2loc, s_o.at[cid].at[sid])


def _compute2(i, sidxa, didxa, h2r, outr, astab, adtab, exbuf, s2loc,
              zero16, jcols, m1):
    def grp(g, c2):
        s16 = sidxa[i, pl.ds(g * 16, 16)]
        d16 = didxa[i, pl.ds(g * 16, 16)]
        asg = plsc.load_gather(astab, [s16])
        adg = plsc.load_gather(adtab, [d16])
        ev = asg + adg
        ev = jnp.maximum(ev, 0.2 * ev)
        ex16 = jnp.exp(ev)
        exbuf[...] = ex16
        for j in range(16):
            e = g * 16 + j
            exb = plsc.load_gather(exbuf, [jcols[j]])
            dv = plsc.load_gather(didxa, [zero16 + i, zero16 + e])
            plsc.addupdate_scatter(s2loc, [dv], exb, mask=m1)
            for k in range(2):
                hv = h2r[e, pl.ds(32 * k, 32)]
                av, bv = plsc.unpack(hv, format=plsc.PackFormat.INTERLEAVED)
                outr[e, pl.ds(32 * k, 16)] = av * exb
                outr[e, pl.ds(32 * k + 16, 16)] = bv * exb
        return c2

    lax.fori_loop(0, B // 16, grp, 0)


def kernel(x, edge_index, W1, a_src1, a_dst1, b1, W2, a_src2, a_dst2, b2):
    f32 = jnp.float32
    i32 = jnp.int32

    e_real = edge_index.shape[1] + N
    e_pad = NW * NBLK1 * B
    assert e_pad >= e_real and NW * NBLK1 == NS * NBLK2
    loop = jnp.arange(N, dtype=i32)
    pad = jnp.full((e_pad - e_real,), N, dtype=i32)
    src = jnp.concatenate([edge_index[0].astype(i32), loop, pad])
    dst = jnp.concatenate([edge_index[1].astype(i32), loop, pad])
    src2d = src.reshape(NW * NBLK1, B)
    dst2d = dst.reshape(NW * NBLK1, B)

    k64 = jnp.arange(HEADS * HID)
    as64 = jnp.zeros((HEADS * HID, HEADS), f32).at[k64, k64 // HID].set(
        a_src1.reshape(-1))
    ad64 = jnp.zeros((HEADS * HID, HEADS), f32).at[k64, k64 // HID].set(
        a_dst1.reshape(-1))
    r8 = jnp.zeros((HEADS, HEADS * HID), f32).at[k64 // HID, k64].set(1.0)
    w1cat = jnp.concatenate(
        [W1, W1 @ as64, jnp.zeros((D_IN, 8), f32)], axis=1)
    w1ad = jnp.concatenate(
        [W1 @ ad64, jnp.zeros((D_IN, 8), f32)], axis=1)
    waa = jnp.concatenate(
        [W2 @ a_src2.reshape(-1, 1), W2 @ a_dst2.reshape(-1, 1),
         jnp.zeros((HEADS * HID, 14), f32)], axis=1)
    ones16 = jnp.ones((NS, 1), f32)
    j64 = jnp.arange(64)
    perm = 32 * (j64 // 32) + (j64 % 2) * 16 + (j64 % 32) // 2
    w2a = W2[:, perm]
    w2b = W2[:, 64 + perm]

    x_pad = jnp.zeros((NPAD, D_IN), f32).at[:N].set(x)

    hs1, ad1 = pl.pallas_call(
        _tc_a,
        out_shape=(jax.ShapeDtypeStruct((NPAD, W1ROW), f32),
                   jax.ShapeDtypeStruct((NPAD, A1ROW), f32)),
    )(x_pad, w1cat, w1ad)

    zm1 = jnp.zeros((NPAD, 72), f32)
    mesh = plsc.VectorSubcoreMesh(core_axis_name="c", subcore_axis_name="s")
    sc_params = pltpu.CompilerParams(
        use_tc_tiling_on_sc=False, needs_layout_passes=False)
    msg1 = pl.kernel(
        _sc_edge1,
        out_type=jax.ShapeDtypeStruct((NC, NPAD, 72), f32),
        mesh=mesh,
        compiler_params=sc_params,
        scratch_types=[
            pltpu.VMEM((PBLK, B), i32),
            pltpu.VMEM((PBLK, B), i32),
            pltpu.VMEM((B, W1ROW), f32),
            pltpu.VMEM((B, W1ROW), f32),
            pltpu.VMEM((B, A1ROW), f32),
            pltpu.VMEM((B, A1ROW), f32),
            pltpu.VMEM((B, 72), f32),
            pltpu.VMEM((B, 72), f32),
            pltpu.VMEM((16,), f32),
            pltpu.VMEM_SHARED((NPAD, 72), f32),
            pltpu.SemaphoreType.DMA,
            pltpu.SemaphoreType.DMA,
        ],
    )(src2d, dst2d, hs1, ad1, zm1)

    hs2a, hs2b, aa = pl.pallas_call(
        _tc_b,
        out_shape=(jax.ShapeDtypeStruct((NPAD, 64), jnp.bfloat16),
                   jax.ShapeDtypeStruct((NPAD, 64), jnp.bfloat16),
                   jax.ShapeDtypeStruct((NPAD, 16), f32)),
    )(msg1[0], msg1[1], r8, w2a, w2b, waa, b1.reshape(1, -1))

    as2 = aa[:, 0].reshape(NPAD)
    ad2 = aa[:, 1].reshape(NPAD)

    zm2 = jnp.zeros((NPAD, 64), f32)
    zs2 = jnp.zeros((NPAD,), f32)
    msg2, s2 = pl.kernel(
        _sc_edge2,
        out_type=(jax.ShapeDtypeStruct((NC, NPAD, 64), f32),
                  jax.ShapeDtypeStruct((NC, NS, NPAD), f32)),
        mesh=mesh,
        compiler_params=sc_params,
        scratch_types=[
            pltpu.VMEM((PBLK, B), i32),
            pltpu.VMEM((PBLK, B), i32),
            pltpu.VMEM((B, 64), jnp.bfloat16),
            pltpu.VMEM((B, 64), jnp.bfloat16),
            pltpu.VMEM((B, 64), f32),
            pltpu.VMEM((B, 64), f32),
            pltpu.VMEM((NPAD,), f32),
            pltpu.VMEM((NPAD,), f32),
            pltpu.VMEM((16,), f32),
            pltpu.VMEM((NPAD,), f32),
            pltpu.VMEM_SHARED((NPAD, 64), f32),
            pltpu.SemaphoreType.DMA,
            pltpu.SemaphoreType.DMA,
        ],
    )(src2d, dst2d, hs2a, hs2b, as2, ad2, zm2, zs2)

    s2pp = s2[0].transpose(1, 0)

    out = pl.pallas_call(
        _tc_c,
        out_shape=jax.ShapeDtypeStruct((NPAD, D_OUT), f32),
    )(msg2[0], msg2[1], s2pp, ones16, b2.reshape(1, -1))
    return out[:N]


# speedup vs baseline: 48.2169x; 1.2379x over previous
import functools

import jax
import jax.numpy as jnp
from jax import lax
from jax.experimental import pallas as pl
from jax.experimental.pallas import tpu as pltpu
from jax.experimental.pallas import tpu_sc as plsc

N = 10000
D_IN = 128
HID = 8
HEADS = 8
D_OUT = 128

NPAD = 10240
NC, NS = 2, 16
NW = NC * NS
RPT = NPAD // NS
B = 128
PBLK = 42
NBLK1 = 2 * PBLK
NBLK2 = 4 * PBLK

W1ROW = 96
A1ROW = 16


def _tc_a(x_ref, w_ref, wa_ref, hs_ref, ad_ref):
    x = x_ref[...]
    hs_ref[...] = jnp.dot(
        x, w_ref[...], preferred_element_type=jnp.float32
    ).astype(jnp.bfloat16)
    ad_ref[...] = jnp.dot(x, wa_ref[...], preferred_element_type=jnp.float32)


def _tc_b(m0_ref, m1_ref, r8_ref, w2a_ref, w2b_ref, waa_ref, b1_ref,
          hs2a_ref, hs2b_ref, aa_ref):
    acc = m0_ref[...] + m1_ref[...]
    srep = jnp.dot(acc[:, 64:72], r8_ref[...],
                   preferred_element_type=jnp.float32)
    g = acc[:, 0:64] / (srep + 1e-16) + b1_ref[...]
    g = jnp.where(g > 0, g, jnp.exp(g) - 1.0)
    hs2a_ref[...] = jnp.dot(
        g, w2a_ref[...], preferred_element_type=jnp.float32
    ).astype(jnp.bfloat16)
    hs2b_ref[...] = jnp.dot(
        g, w2b_ref[...], preferred_element_type=jnp.float32
    ).astype(jnp.bfloat16)
    aa_ref[...] = jnp.dot(g, waa_ref[...], preferred_element_type=jnp.float32)


def _tc_c(m0_ref, m1_ref, sp_ref, ones_ref, b2_ref, out_ref):
    s = jnp.dot(sp_ref[...], ones_ref[...],
                preferred_element_type=jnp.float32)
    v = (jnp.concatenate([m0_ref[...], m1_ref[...]], axis=1) / (s + 1e-16)
         + b2_ref[...])
    m = jnp.max(v, axis=1, keepdims=True)
    u = v - m
    lse = jnp.log(jnp.sum(jnp.exp(u), axis=1, keepdims=True))
    out_ref[...] = u - lse


def _sc_edge1(src_h, dst_h, hs_h, ad_h, zm_h, msg_o,
              sidxa, didxa, hsr0, hsr1, adr0, adr1, outr0, outr1,
              exbuf, accm, gs0, gs1):
    cid = lax.axis_index("c")
    sid = lax.axis_index("s")
    wid = sid * NC + cid
    pltpu.sync_copy(zm_h.at[pl.ds(sid * RPT, RPT)],
                    accm.at[pl.ds(sid * RPT, RPT)])
    plsc.subcore_barrier()
    lane = lax.broadcasted_iota(jnp.int32, (16,), 0)
    half = lane // 8
    zero16 = lane // 16
    cols = [half + 2 * k for k in range(4)]
    lane7 = jnp.bitwise_and(lane, 7)
    m8 = lane < 8

    def gathers(i, hsr, adr, gsem):
        a = pltpu.async_copy(hs_h.at[sidxa.at[i]], hsr, gsem)
        b = pltpu.async_copy(ad_h.at[didxa.at[i]], adr, gsem)
        return a, b

    def drain(i, hsr, adr, gsem):
        pltpu.make_async_copy(hs_h.at[sidxa.at[i]], hsr, gsem).wait()
        pltpu.make_async_copy(ad_h.at[didxa.at[i]], adr, gsem).wait()

    def compute(hsr, adr, outr):
        def edge(e, c2):
            as32 = hsr[e, pl.ds(64, 32)]
            asv, _ = plsc.unpack(as32, format=plsc.PackFormat.INTERLEAVED)
            adv = adr[e, :]
            ev = asv + adv
            ev = jnp.maximum(ev, 0.2 * ev)
            ex = jnp.exp(ev)
            exbuf[...] = ex
            rowi = zero16 + e
            plsc.store_scatter(outr, [rowi, 64 + lane7], ex, mask=m8)
            for k in range(2):
                hv = hsr[e, pl.ds(32 * k, 32)]
                av, bv = plsc.unpack(hv, format=plsc.PackFormat.INTERLEAVED)
                exba = plsc.load_gather(exbuf, [cols[2 * k]])
                exbb = plsc.load_gather(exbuf, [cols[2 * k + 1]])
                outr[e, pl.ds(32 * k, 16)] = av * exba
                outr[e, pl.ds(32 * k + 16, 16)] = bv * exbb
            return c2

        lax.fori_loop(0, B, edge, 0, unroll=2)

    def scatter(i, outr):
        pltpu.sync_copy(outr, accm.at[didxa.at[i]], add=True)

    for p in range(NBLK1 // PBLK):
        row0 = wid * NBLK1 + p * PBLK
        pltpu.sync_copy(src_h.at[pl.ds(row0, PBLK)], sidxa)
        pltpu.sync_copy(dst_h.at[pl.ds(row0, PBLK)], didxa)
        gathers(0, hsr0, adr0, gs0)

        def body(j, carry):
            i0 = 2 * j
            i1 = i0 + 1
            c1a, c1b = gathers(i1, hsr1, adr1, gs1)
            drain(i0, hsr0, adr0, gs0)
            compute(hsr0, adr0, outr0)

            @pl.when(i0 + 2 < PBLK)
            def _():
                gathers(i0 + 2, hsr0, adr0, gs0)

            scatter(i0, outr0)
            c1a.wait()
            c1b.wait()
            compute(hsr1, adr1, outr1)

            @pl.when(i1 + 2 < PBLK)
            def _():
                gathers(i1 + 2, hsr1, adr1, gs1)

            scatter(i1, outr1)
            return carry

        lax.fori_loop(0, PBLK // 2, body, 0)

    plsc.subcore_barrier()
    pltpu.sync_copy(accm.at[pl.ds(sid * RPT, RPT)],
                    msg_o.at[cid].at[pl.ds(sid * RPT, RPT)])


def _sc_edge2(src_h, dst_h, hs2a_h, hs2b_h, as2_h, ad2_h, zm_h, zs_h,
              msg_o, s_o,
              sidxa, didxa, h2r0, h2r1, outr0, outr1, astab, adtab, exbuf,
              s2loc, accm, gs0, gs1):
    cid = lax.axis_index("c")
    sid = lax.axis_index("s")
    pltpu.sync_copy(as2_h, astab)
    pltpu.sync_copy(ad2_h, adtab)
    pltpu.sync_copy(zs_h, s2loc)
    pltpu.sync_copy(zm_h.at[pl.ds(sid * RPT, RPT)],
                    accm.at[pl.ds(sid * RPT, RPT)])
    plsc.subcore_barrier()
    lane = lax.broadcasted_iota(jnp.int32, (16,), 0)
    zero16 = lane // 16
    jcols = [zero16 + j for j in range(16)]
    m1 = lane < 1

    def gathers(i, h2r, gsem):
        @pl.when(cid == 0)
        def _():
            pltpu.async_copy(hs2a_h.at[sidxa.at[i]], h2r, gsem)

        @pl.when(cid == 1)
        def _():
            pltpu.async_copy(hs2b_h.at[sidxa.at[i]], h2r, gsem)

    def drain(i, h2r, gsem):
        pltpu.make_async_copy(hs2a_h.at[sidxa.at[i]], h2r, gsem).wait()

    def scatter(i, outr):
        pltpu.sync_copy(outr, accm.at[didxa.at[i]], add=True)

    for p in range(NBLK2 // PBLK):
        row0 = sid * NBLK2 + p * PBLK
        pltpu.sync_copy(src_h.at[pl.ds(row0, PBLK)], sidxa)
        pltpu.sync_copy(dst_h.at[pl.ds(row0, PBLK)], didxa)
        gathers(0, h2r0, gs0)

        def body(j, carry):
            i0 = 2 * j
            i1 = i0 + 1
            gathers(i1, h2r1, gs1)
            drain(i0, h2r0, gs0)
            _compute2(i0, sidxa, didxa, h2r0, outr0, astab, adtab, exbuf,
                      s2loc, zero16, jcols, m1)

            @pl.when(i0 + 2 < PBLK)
            def _():
                gathers(i0 + 2, h2r0, gs0)

            scatter(i0, outr0)
            drain(i1, h2r1, gs1)
            _compute2(i1, sidxa, didxa, h2r1, outr1, astab, adtab, exbuf,
                      s2loc, zero16, jcols, m1)

            @pl.when(i1 + 2 < PBLK)
            def _():
                gathers(i1 + 2, h2r1, gs1)

            scatter(i1, outr1)
            return carry

        lax.fori_loop(0, PBLK // 2, body, 0)

    plsc.subcore_barrier()
    pltpu.sync_copy(accm.at[pl.ds(sid * RPT, RPT)],
                    msg_o.at[cid].at[pl.ds(sid * RPT, RPT)])
    pltpu.sync_copy(s2loc, s_o.at[cid].at[sid])


def _compute2(i, sidxa, didxa, h2r, outr, astab, adtab, exbuf, s2loc,
              zero16, jcols, m1):
    def grp(g, c2):
        s16 = sidxa[i, pl.ds(g * 16, 16)]
        d16 = didxa[i, pl.ds(g * 16, 16)]
        asg = plsc.load_gather(astab, [s16])
        adg = plsc.load_gather(adtab, [d16])
        ev = asg + adg
        ev = jnp.maximum(ev, 0.2 * ev)
        ex16 = jnp.exp(ev)
        exbuf[...] = ex16
        for j in range(16):
            e = g * 16 + j
            exb = plsc.load_gather(exbuf, [jcols[j]])
            dv = plsc.load_gather(didxa, [zero16 + i, zero16 + e])
            plsc.addupdate_scatter(s2loc, [dv], exb, mask=m1)
            for k in range(2):
                hv = h2r[e, pl.ds(32 * k, 32)]
                av, bv = plsc.unpack(hv, format=plsc.PackFormat.INTERLEAVED)
                outr[e, pl.ds(32 * k, 16)] = av * exb
                outr[e, pl.ds(32 * k + 16, 16)] = bv * exb
        return c2

    lax.fori_loop(0, B // 16, grp, 0)


def kernel(x, edge_index, W1, a_src1, a_dst1, b1, W2, a_src2, a_dst2, b2):
    f32 = jnp.float32
    i32 = jnp.int32

    e_real = edge_index.shape[1] + N
    e_pad = NW * NBLK1 * B
    assert e_pad >= e_real and NW * NBLK1 == NS * NBLK2
    loop = jnp.arange(N, dtype=i32)
    pad = jnp.full((e_pad - e_real,), N, dtype=i32)
    src = jnp.concatenate([edge_index[0].astype(i32), loop, pad])
    dst = jnp.concatenate([edge_index[1].astype(i32), loop, pad])
    src2d = src.reshape(NW * NBLK1, B)
    dst2d = dst.reshape(NW * NBLK1, B)

    k64 = jnp.arange(HEADS * HID)
    as64 = jnp.zeros((HEADS * HID, HEADS), f32).at[k64, k64 // HID].set(
        a_src1.reshape(-1))
    ad64 = jnp.zeros((HEADS * HID, HEADS), f32).at[k64, k64 // HID].set(
        a_dst1.reshape(-1))
    r8 = jnp.zeros((HEADS, HEADS * HID), f32).at[k64 // HID, k64].set(1.0)
    j64 = jnp.arange(64)
    perm = 32 * (j64 // 32) + (j64 % 2) * 16 + (j64 % 32) // 2
    asport = jnp.zeros((D_IN, 32), f32).at[:, 2 * jnp.arange(8)].set(
        W1 @ as64)
    w1cat = jnp.concatenate([W1[:, perm], asport], axis=1)
    w1ad = jnp.concatenate(
        [W1 @ ad64, jnp.zeros((D_IN, 8), f32)], axis=1)
    waa = jnp.concatenate(
        [W2 @ a_src2.reshape(-1, 1), W2 @ a_dst2.reshape(-1, 1),
         jnp.zeros((HEADS * HID, 14), f32)], axis=1)
    ones16 = jnp.ones((NS, 1), f32)
    w2a = W2[:, perm]
    w2b = W2[:, 64 + perm]

    x_pad = jnp.zeros((NPAD, D_IN), f32).at[:N].set(x)

    hs1, ad1 = pl.pallas_call(
        _tc_a,
        out_shape=(jax.ShapeDtypeStruct((NPAD, W1ROW), jnp.bfloat16),
                   jax.ShapeDtypeStruct((NPAD, A1ROW), f32)),
    )(x_pad, w1cat, w1ad)

    zm1 = jnp.zeros((NPAD, 72), f32)
    mesh = plsc.VectorSubcoreMesh(core_axis_name="c", subcore_axis_name="s")
    sc_params = pltpu.CompilerParams(
        use_tc_tiling_on_sc=False, needs_layout_passes=False)
    msg1 = pl.kernel(
        _sc_edge1,
        out_type=jax.ShapeDtypeStruct((NC, NPAD, 72), f32),
        mesh=mesh,
        compiler_params=sc_params,
        scratch_types=[
            pltpu.VMEM((PBLK, B), i32),
            pltpu.VMEM((PBLK, B), i32),
            pltpu.VMEM((B, W1ROW), jnp.bfloat16),
            pltpu.VMEM((B, W1ROW), jnp.bfloat16),
            pltpu.VMEM((B, A1ROW), f32),
            pltpu.VMEM((B, A1ROW), f32),
            pltpu.VMEM((B, 72), f32),
            pltpu.VMEM((B, 72), f32),
            pltpu.VMEM((16,), f32),
            pltpu.VMEM_SHARED((NPAD, 72), f32),
            pltpu.SemaphoreType.DMA,
            pltpu.SemaphoreType.DMA,
        ],
    )(src2d, dst2d, hs1, ad1, zm1)

    hs2a, hs2b, aa = pl.pallas_call(
        _tc_b,
        out_shape=(jax.ShapeDtypeStruct((NPAD, 64), jnp.bfloat16),
                   jax.ShapeDtypeStruct((NPAD, 64), jnp.bfloat16),
                   jax.ShapeDtypeStruct((NPAD, 16), f32)),
    )(msg1[0], msg1[1], r8, w2a, w2b, waa, b1.reshape(1, -1))

    as2 = aa[:, 0].reshape(NPAD)
    ad2 = aa[:, 1].reshape(NPAD)

    zm2 = jnp.zeros((NPAD, 64), f32)
    zs2 = jnp.zeros((NPAD,), f32)
    msg2, s2 = pl.kernel(
        _sc_edge2,
        out_type=(jax.ShapeDtypeStruct((NC, NPAD, 64), f32),
                  jax.ShapeDtypeStruct((NC, NS, NPAD), f32)),
        mesh=mesh,
        compiler_params=sc_params,
        scratch_types=[
            pltpu.VMEM((PBLK, B), i32),
            pltpu.VMEM((PBLK, B), i32),
            pltpu.VMEM((B, 64), jnp.bfloat16),
            pltpu.VMEM((B, 64), jnp.bfloat16),
            pltpu.VMEM((B, 64), f32),
            pltpu.VMEM((B, 64), f32),
            pltpu.VMEM((NPAD,), f32),
            pltpu.VMEM((NPAD,), f32),
            pltpu.VMEM((16,), f32),
            pltpu.VMEM((NPAD,), f32),
            pltpu.VMEM_SHARED((NPAD, 64), f32),
            pltpu.SemaphoreType.DMA,
            pltpu.SemaphoreType.DMA,
        ],
    )(src2d, dst2d, hs2a, hs2b, as2, ad2, zm2, zs2)

    s2pp = s2[0].transpose(1, 0)

    out = pl.pallas_call(
        _tc_c,
        out_shape=jax.ShapeDtypeStruct((NPAD, D_OUT), f32),
    )(msg2[0], msg2[1], s2pp, ones16, b2.reshape(1, -1))
    return out[:N]


# speedup vs baseline: 48.8622x; 1.0134x over previous
import functools

import jax
import jax.numpy as jnp
from jax import lax
from jax.experimental import pallas as pl
from jax.experimental.pallas import tpu as pltpu
from jax.experimental.pallas import tpu_sc as plsc

N = 10000
D_IN = 128
HID = 8
HEADS = 8
D_OUT = 128

NPAD = 10240
NC, NS = 2, 16
NW = NC * NS
RPT = NPAD // NS
B = 128
B2 = 64
PBLK = 28
NBLK1 = 3 * PBLK
NBLK2 = 6 * PBLK

W1ROW = 96
A1ROW = 16


def _tc_a(x_ref, w_ref, wa_ref, hs_ref, ad_ref):
    x = x_ref[...]
    hs_ref[...] = jnp.dot(
        x, w_ref[...], preferred_element_type=jnp.float32
    ).astype(jnp.bfloat16)
    ad_ref[...] = jnp.dot(x, wa_ref[...], preferred_element_type=jnp.float32)


def _tc_b(m0_ref, m1_ref, r8_ref, w2p_ref, waa_ref, b1_ref,
          hs2_ref, aa_ref):
    acc = m0_ref[...] + m1_ref[...]
    srep = jnp.dot(acc[:, 64:72], r8_ref[...],
                   preferred_element_type=jnp.float32)
    g = acc[:, 0:64] / (srep + 1e-16) + b1_ref[...]
    g = jnp.where(g > 0, g, jnp.exp(g) - 1.0)
    hs2_ref[...] = jnp.dot(
        g, w2p_ref[...], preferred_element_type=jnp.float32
    ).astype(jnp.bfloat16)
    aa_ref[...] = jnp.dot(g, waa_ref[...], preferred_element_type=jnp.float32)


def _tc_c(m0_ref, m1_ref, sp_ref, ones_ref, b2_ref, out_ref):
    s = lax.dot_general(sp_ref[...], ones_ref[...],
                        (((0,), (0,)), ((), ())),
                        preferred_element_type=jnp.float32)
    v = (m0_ref[...] + m1_ref[...]) / (s + 1e-16) + b2_ref[...]
    m = jnp.max(v, axis=1, keepdims=True)
    u = v - m
    lse = jnp.log(jnp.sum(jnp.exp(u), axis=1, keepdims=True))
    out_ref[...] = u - lse


def _sc_edge1(src_h, dst_h, hs_h, ad_h, zm_h, msg_o,
              sidxa, didxa, hsr0, hsr1, adr0, adr1, outr0, outr1,
              exbuf, accm, gs0, gs1):
    cid = lax.axis_index("c")
    sid = lax.axis_index("s")
    wid = sid * NC + cid
    pltpu.sync_copy(zm_h.at[pl.ds(sid * RPT, RPT)],
                    accm.at[pl.ds(sid * RPT, RPT)])
    plsc.subcore_barrier()
    lane = lax.broadcasted_iota(jnp.int32, (16,), 0)
    half = lane // 8
    zero16 = lane // 16
    cols = [half + 2 * k for k in range(4)]
    lane7 = jnp.bitwise_and(lane, 7)
    m8 = lane < 8

    def gathers(i, hsr, adr, gsem):
        a = pltpu.async_copy(hs_h.at[sidxa.at[i]], hsr, gsem)
        b = pltpu.async_copy(ad_h.at[didxa.at[i]], adr, gsem)
        return a, b

    def drain(i, hsr, adr, gsem):
        pltpu.make_async_copy(hs_h.at[sidxa.at[i]], hsr, gsem).wait()
        pltpu.make_async_copy(ad_h.at[didxa.at[i]], adr, gsem).wait()

    def compute(hsr, adr, outr):
        def edge(e, c2):
            as32 = hsr[e, pl.ds(64, 32)]
            asv, _ = plsc.unpack(as32, format=plsc.PackFormat.INTERLEAVED)
            adv = adr[e, :]
            ev = asv + adv
            ev = jnp.maximum(ev, 0.2 * ev)
            ex = jnp.exp(ev)
            exbuf[...] = ex
            rowi = zero16 + e
            plsc.store_scatter(outr, [rowi, 64 + lane7], ex, mask=m8)
            for k in range(2):
                hv = hsr[e, pl.ds(32 * k, 32)]
                av, bv = plsc.unpack(hv, format=plsc.PackFormat.INTERLEAVED)
                exba = plsc.load_gather(exbuf, [cols[2 * k]])
                exbb = plsc.load_gather(exbuf, [cols[2 * k + 1]])
                outr[e, pl.ds(32 * k, 16)] = av * exba
                outr[e, pl.ds(32 * k + 16, 16)] = bv * exbb
            return c2

        lax.fori_loop(0, B, edge, 0, unroll=2)

    def scatter(i, outr):
        pltpu.sync_copy(outr, accm.at[didxa.at[i]], add=True)

    for p in range(NBLK1 // PBLK):
        row0 = wid * NBLK1 + p * PBLK
        pltpu.sync_copy(src_h.at[pl.ds(row0, PBLK)], sidxa)
        pltpu.sync_copy(dst_h.at[pl.ds(row0, PBLK)], didxa)
        gathers(0, hsr0, adr0, gs0)

        def body(j, carry):
            i0 = 2 * j
            i1 = i0 + 1
            c1a, c1b = gathers(i1, hsr1, adr1, gs1)
            drain(i0, hsr0, adr0, gs0)
            compute(hsr0, adr0, outr0)

            @pl.when(i0 + 2 < PBLK)
            def _():
                gathers(i0 + 2, hsr0, adr0, gs0)

            scatter(i0, outr0)
            c1a.wait()
            c1b.wait()
            compute(hsr1, adr1, outr1)

            @pl.when(i1 + 2 < PBLK)
            def _():
                gathers(i1 + 2, hsr1, adr1, gs1)

            scatter(i1, outr1)
            return carry

        lax.fori_loop(0, PBLK // 2, body, 0)

    plsc.subcore_barrier()
    pltpu.sync_copy(accm.at[pl.ds(sid * RPT, RPT)],
                    msg_o.at[cid].at[pl.ds(sid * RPT, RPT)])


def _sc_edge2(src_h, dst_h, hs2_h, aspk_h, zm_h, zs_h, msg_o, s_o,
              sidxa, didxa, h2r0, h2r1, outr0, outr1, aspk, exbuf,
              s2loc, accm, gs0, gs1):
    cid = lax.axis_index("c")
    sid = lax.axis_index("s")
    wid = sid * NC + cid
    pltpu.sync_copy(aspk_h, aspk)
    pltpu.sync_copy(zs_h, s2loc)
    pltpu.sync_copy(zm_h.at[pl.ds(sid * RPT, RPT)],
                    accm.at[pl.ds(sid * RPT, RPT)])
    plsc.subcore_barrier()
    lane = lax.broadcasted_iota(jnp.int32, (16,), 0)
    zero16 = lane // 16
    jcols = [zero16 + j for j in range(16)]
    m1 = lane < 1

    def gathers(i, h2r, gsem):
        pltpu.async_copy(hs2_h.at[sidxa.at[i]], h2r, gsem)

    def drain(i, h2r, gsem):
        pltpu.make_async_copy(hs2_h.at[sidxa.at[i]], h2r, gsem).wait()

    def compute(i, h2r, outr):
        def grp(g, c2):
            s16 = sidxa[i, pl.ds(g * 16, 16)]
            d16 = didxa[i, pl.ds(g * 16, 16)]
            asg, _ = plsc.unpack(
                plsc.bitcast(plsc.load_gather(aspk, [s16]), jnp.bfloat16),
                format=plsc.PackFormat.INTERLEAVED)
            _, adg = plsc.unpack(
                plsc.bitcast(plsc.load_gather(aspk, [d16]), jnp.bfloat16),
                format=plsc.PackFormat.INTERLEAVED)
            ev = asg + adg
            ev = jnp.maximum(ev, 0.2 * ev)
            ex16 = jnp.exp(ev)
            exbuf[...] = ex16
            for j in range(16):
                e = g * 16 + j
                exb = plsc.load_gather(exbuf, [jcols[j]])
                dv = plsc.load_gather(didxa, [zero16 + i, zero16 + e])
                plsc.addupdate_scatter(s2loc, [dv], exb, mask=m1)
                for k in range(4):
                    hv = h2r[e, pl.ds(32 * k, 32)]
                    av, bv = plsc.unpack(
                        hv, format=plsc.PackFormat.INTERLEAVED)
                    outr[e, pl.ds(32 * k, 16)] = av * exb
                    outr[e, pl.ds(32 * k + 16, 16)] = bv * exb
            return c2

        lax.fori_loop(0, B2 // 16, grp, 0)

    def scatter(i, outr):
        pltpu.sync_copy(outr, accm.at[didxa.at[i]], add=True)

    for p in range(NBLK2 // PBLK):
        row0 = wid * NBLK2 + p * PBLK
        pltpu.sync_copy(src_h.at[pl.ds(row0, PBLK)], sidxa)
        pltpu.sync_copy(dst_h.at[pl.ds(row0, PBLK)], didxa)
        gathers(0, h2r0, gs0)

        def body(j, carry):
            i0 = 2 * j
            i1 = i0 + 1
            gathers(i1, h2r1, gs1)
            drain(i0, h2r0, gs0)
            compute(i0, h2r0, outr0)

            @pl.when(i0 + 2 < PBLK)
            def _():
                gathers(i0 + 2, h2r0, gs0)

            scatter(i0, outr0)
            drain(i1, h2r1, gs1)
            compute(i1, h2r1, outr1)

            @pl.when(i1 + 2 < PBLK)
            def _():
                gathers(i1 + 2, h2r1, gs1)

            scatter(i1, outr1)
            return carry

        lax.fori_loop(0, PBLK // 2, body, 0)

    plsc.subcore_barrier()
    pltpu.sync_copy(accm.at[pl.ds(sid * RPT, RPT)],
                    msg_o.at[cid].at[pl.ds(sid * RPT, RPT)])
    pltpu.sync_copy(s2loc, s_o.at[cid].at[sid])


def kernel(x, edge_index, W1, a_src1, a_dst1, b1, W2, a_src2, a_dst2, b2):
    f32 = jnp.float32
    i32 = jnp.int32

    e_real = edge_index.shape[1] + N
    e_pad = NW * NBLK1 * B
    assert e_pad >= e_real and NW * NBLK1 == NS * NBLK2
    loop = jnp.arange(N, dtype=i32)
    pad = jnp.full((e_pad - e_real,), N, dtype=i32)
    src = jnp.concatenate([edge_index[0].astype(i32), loop, pad])
    dst = jnp.concatenate([edge_index[1].astype(i32), loop, pad])
    src2d = src.reshape(NW * NBLK1, B)
    dst2d = dst.reshape(NW * NBLK1, B)
    src2d64 = src.reshape(NW * NBLK2, B2)
    dst2d64 = dst.reshape(NW * NBLK2, B2)

    k64 = jnp.arange(HEADS * HID)
    as64 = jnp.zeros((HEADS * HID, HEADS), f32).at[k64, k64 // HID].set(
        a_src1.reshape(-1))
    ad64 = jnp.zeros((HEADS * HID, HEADS), f32).at[k64, k64 // HID].set(
        a_dst1.reshape(-1))
    r8 = jnp.zeros((HEADS, HEADS * HID), f32).at[k64 // HID, k64].set(1.0)
    j64 = jnp.arange(64)
    perm = 32 * (j64 // 32) + (j64 % 2) * 16 + (j64 % 32) // 2
    asport = jnp.zeros((D_IN, 32), f32).at[:, 2 * jnp.arange(8)].set(
        W1 @ as64)
    w1cat = jnp.concatenate([W1[:, perm], asport], axis=1)
    w1ad = jnp.concatenate(
        [W1 @ ad64, jnp.zeros((D_IN, 8), f32)], axis=1)
    waa = jnp.concatenate(
        [W2 @ a_src2.reshape(-1, 1), W2 @ a_dst2.reshape(-1, 1),
         jnp.zeros((HEADS * HID, 14), f32)], axis=1)
    ones32 = jnp.ones((NW, 1), f32)
    j128 = jnp.arange(128)
    perm128 = 32 * (j128 // 32) + (j128 % 2) * 16 + (j128 % 32) // 2
    w2p = W2[:, perm128]

    x_pad = jnp.zeros((NPAD, D_IN), f32).at[:N].set(x)

    hs1, ad1 = pl.pallas_call(
        _tc_a,
        out_shape=(jax.ShapeDtypeStruct((NPAD, W1ROW), jnp.bfloat16),
                   jax.ShapeDtypeStruct((NPAD, A1ROW), f32)),
    )(x_pad, w1cat, w1ad)

    zm1 = jnp.zeros((NPAD, 72), f32)
    mesh = plsc.VectorSubcoreMesh(core_axis_name="c", subcore_axis_name="s")
    sc_params = pltpu.CompilerParams(
        use_tc_tiling_on_sc=False, needs_layout_passes=False)
    msg1 = pl.kernel(
        _sc_edge1,
        out_type=jax.ShapeDtypeStruct((NC, NPAD, 72), f32),
        mesh=mesh,
        compiler_params=sc_params,
        scratch_types=[
            pltpu.VMEM((PBLK, B), i32),
            pltpu.VMEM((PBLK, B), i32),
            pltpu.VMEM((B, W1ROW), jnp.bfloat16),
            pltpu.VMEM((B, W1ROW), jnp.bfloat16),
            pltpu.VMEM((B, A1ROW), f32),
            pltpu.VMEM((B, A1ROW), f32),
            pltpu.VMEM((B, 72), f32),
            pltpu.VMEM((B, 72), f32),
            pltpu.VMEM((16,), f32),
            pltpu.VMEM_SHARED((NPAD, 72), f32),
            pltpu.SemaphoreType.DMA,
            pltpu.SemaphoreType.DMA,
        ],
    )(src2d, dst2d, hs1, ad1, zm1)

    hs2, aa = pl.pallas_call(
        _tc_b,
        out_shape=(jax.ShapeDtypeStruct((NPAD, D_OUT), jnp.bfloat16),
                   jax.ShapeDtypeStruct((NPAD, 16), f32)),
    )(msg1[0], msg1[1], r8, w2p, waa, b1.reshape(1, -1))

    as2u = lax.bitcast_convert_type(
        aa[:, 0].astype(jnp.bfloat16), jnp.uint16).astype(jnp.uint32)
    ad2u = lax.bitcast_convert_type(
        aa[:, 1].astype(jnp.bfloat16), jnp.uint16).astype(jnp.uint32)
    aspk = lax.bitcast_convert_type(
        (ad2u << 16) | as2u, jnp.int32).reshape(NPAD)

    zm2 = jnp.zeros((NPAD, D_OUT), f32)
    zs2 = jnp.zeros((NPAD,), f32)
    msg2, s2 = pl.kernel(
        _sc_edge2,
        out_type=(jax.ShapeDtypeStruct((NC, NPAD, D_OUT), f32),
                  jax.ShapeDtypeStruct((NC, NS, NPAD), f32)),
        mesh=mesh,
        compiler_params=sc_params,
        scratch_types=[
            pltpu.VMEM((PBLK, B2), i32),
            pltpu.VMEM((PBLK, B2), i32),
            pltpu.VMEM((B2, D_OUT), jnp.bfloat16),
            pltpu.VMEM((B2, D_OUT), jnp.bfloat16),
            pltpu.VMEM((B2, D_OUT), f32),
            pltpu.VMEM((B2, D_OUT), f32),
            pltpu.VMEM((NPAD,), i32),
            pltpu.VMEM((16,), f32),
            pltpu.VMEM((NPAD,), f32),
            pltpu.VMEM_SHARED((NPAD, D_OUT), f32),
            pltpu.SemaphoreType.DMA,
            pltpu.SemaphoreType.DMA,
        ],
    )(src2d64, dst2d64, hs2, aspk, zm2, zs2)

    out = pl.pallas_call(
        _tc_c,
        out_shape=jax.ShapeDtypeStruct((NPAD, D_OUT), f32),
    )(msg2[0], msg2[1], s2.reshape(NW, NPAD), ones32, b2.reshape(1, -1))
    return out[:N]


# speedup vs baseline: 55.3548x; 1.1329x over previous
import functools

import jax
import jax.numpy as jnp
from jax import lax
from jax.experimental import pallas as pl
from jax.experimental.pallas import tpu as pltpu
from jax.experimental.pallas import tpu_sc as plsc

N = 10000
D_IN = 128
HID = 8
HEADS = 8
D_OUT = 128

NPAD = 10240
NC, NS = 2, 16
NW = NC * NS
RPT = NPAD // NS
B = 128
B2 = 64
PBLK1 = 40
PBLK2 = 20
NBLK1 = 2 * PBLK1
NBLK2 = 8 * PBLK2

W1ROW = 96
A1ROW = 16


def _tc_a(x_ref, w_ref, wa_ref, w1_ref, w1as_ref, r8_ref,
          hs_ref, ad_ref, init_ref):
    x = x_ref[...]
    hs_ref[...] = jnp.dot(
        x, w_ref[...], preferred_element_type=jnp.float32
    ).astype(jnp.bfloat16)
    ad = jnp.dot(x, wa_ref[...], preferred_element_type=jnp.float32)
    ad_ref[...] = ad
    h1 = jnp.dot(x, w1_ref[...], preferred_element_type=jnp.float32)
    ev = (jnp.dot(x, w1as_ref[...], preferred_element_type=jnp.float32)
          + ad[:, 0:8])
    ev = jnp.maximum(ev, 0.2 * ev)
    exs = 0.5 * jnp.exp(ev)
    exrep = jnp.dot(exs, r8_ref[...], preferred_element_type=jnp.float32)
    init_ref[:, 0:64] = h1 * exrep
    init_ref[:, 64:72] = exs


def _tc_b(m0_ref, m1_ref, r8_ref, w2p_ref, w2_ref, waa_ref, b1_ref,
          hs2_ref, aa_ref, init_ref):
    acc = m0_ref[...] + m1_ref[...]
    srep = jnp.dot(acc[:, 64:72], r8_ref[...],
                   preferred_element_type=jnp.float32)
    g = acc[:, 0:64] / (srep + 1e-16) + b1_ref[...]
    g = jnp.where(g > 0, g, jnp.exp(g) - 1.0)
    hs2_ref[...] = jnp.dot(
        g, w2p_ref[...], preferred_element_type=jnp.float32
    ).astype(jnp.bfloat16)
    aav = jnp.dot(g, waa_ref[...], preferred_element_type=jnp.float32)
    ev = aav[:, 0:1] + aav[:, 1:2]
    ev = jnp.maximum(ev, 0.2 * ev)
    exs2 = jnp.exp(ev)
    aa_ref[...] = jnp.concatenate(
        [aav[:, 0:2], exs2, aav[:, 3:16]], axis=1)
    h2n = jnp.dot(g, w2_ref[...], preferred_element_type=jnp.float32)
    init_ref[...] = h2n * (0.5 * exs2)


def _tc_c(m0_ref, m1_ref, sp_ref, ones_ref, aa_ref, b2_ref, out_ref):
    s = lax.dot_general(sp_ref[...], ones_ref[...],
                        (((0,), (0,)), ((), ())),
                        preferred_element_type=jnp.float32)
    s = s + aa_ref[:, 2:3]
    v = (m0_ref[...] + m1_ref[...]) / (s + 1e-16) + b2_ref[...]
    m = jnp.max(v, axis=1, keepdims=True)
    u = v - m
    lse = jnp.log(jnp.sum(jnp.exp(u), axis=1, keepdims=True))
    out_ref[...] = u - lse


def _sc_edge1(src_h, dst_h, hs_h, ad_h, zm_h, msg_o,
              sidxa, didxa, hsr0, hsr1, adr0, adr1, outr0, outr1,
              exbuf, accm, gs0, gs1):
    cid = lax.axis_index("c")
    sid = lax.axis_index("s")
    wid = sid * NC + cid
    pltpu.sync_copy(zm_h.at[pl.ds(sid * RPT, RPT)],
                    accm.at[pl.ds(sid * RPT, RPT)])
    plsc.subcore_barrier()
    lane = lax.broadcasted_iota(jnp.int32, (16,), 0)
    half = lane // 8
    zero16 = lane // 16
    cols = [half + 2 * k for k in range(4)]
    lane7 = jnp.bitwise_and(lane, 7)
    m8 = lane < 8

    def gathers(i, hsr, adr, gsem):
        a = pltpu.async_copy(hs_h.at[sidxa.at[i]], hsr, gsem)
        b = pltpu.async_copy(ad_h.at[didxa.at[i]], adr, gsem)
        return a, b

    def drain(i, hsr, adr, gsem):
        pltpu.make_async_copy(hs_h.at[sidxa.at[i]], hsr, gsem).wait()
        pltpu.make_async_copy(ad_h.at[didxa.at[i]], adr, gsem).wait()

    def compute(hsr, adr, outr):
        def edge(e, c2):
            as32 = hsr[e, pl.ds(64, 32)]
            asv, _ = plsc.unpack(as32, format=plsc.PackFormat.INTERLEAVED)
            adv = adr[e, :]
            ev = asv + adv
            ev = jnp.maximum(ev, 0.2 * ev)
            ex = jnp.exp(ev)
            exbuf[...] = ex
            rowi = zero16 + e
            plsc.store_scatter(outr, [rowi, 64 + lane7], ex, mask=m8)
            for k in range(2):
                hv = hsr[e, pl.ds(32 * k, 32)]
                av, bv = plsc.unpack(hv, format=plsc.PackFormat.INTERLEAVED)
                exba = plsc.load_gather(exbuf, [cols[2 * k]])
                exbb = plsc.load_gather(exbuf, [cols[2 * k + 1]])
                outr[e, pl.ds(32 * k, 16)] = av * exba
                outr[e, pl.ds(32 * k + 16, 16)] = bv * exbb
            return c2

        lax.fori_loop(0, B, edge, 0, unroll=2)

    def scatter(i, outr):
        pltpu.sync_copy(outr, accm.at[didxa.at[i]], add=True)

    def phase(p, pcarry):
        row0 = wid * NBLK1 + p * PBLK1
        pltpu.sync_copy(src_h.at[pl.ds(row0, PBLK1)], sidxa)
        pltpu.sync_copy(dst_h.at[pl.ds(row0, PBLK1)], didxa)
        gathers(0, hsr0, adr0, gs0)

        def body(j, carry):
            i0 = 2 * j
            i1 = i0 + 1
            c1a, c1b = gathers(i1, hsr1, adr1, gs1)
            drain(i0, hsr0, adr0, gs0)
            compute(hsr0, adr0, outr0)

            @pl.when(i0 + 2 < PBLK1)
            def _():
                gathers(i0 + 2, hsr0, adr0, gs0)

            scatter(i0, outr0)
            c1a.wait()
            c1b.wait()
            compute(hsr1, adr1, outr1)

            @pl.when(i1 + 2 < PBLK1)
            def _():
                gathers(i1 + 2, hsr1, adr1, gs1)

            scatter(i1, outr1)
            return carry

        lax.fori_loop(0, PBLK1 // 2, body, 0)
        return pcarry

    lax.fori_loop(0, NBLK1 // PBLK1, phase, 0)
    plsc.subcore_barrier()
    pltpu.sync_copy(accm.at[pl.ds(sid * RPT, RPT)],
                    msg_o.at[cid].at[pl.ds(sid * RPT, RPT)])


def _sc_edge2(src_h, dst_h, hs2_h, aspk_h, zm_h, zs_h, msg_o, s_o,
              sidxa, didxa, h2r0, h2r1, outr0, outr1, aspk, exbuf,
              s2loc, accm, gs0, gs1):
    cid = lax.axis_index("c")
    sid = lax.axis_index("s")
    wid = sid * NC + cid
    pltpu.sync_copy(aspk_h, aspk)
    pltpu.sync_copy(zs_h, s2loc)
    pltpu.sync_copy(zm_h.at[pl.ds(sid * RPT, RPT)],
                    accm.at[pl.ds(sid * RPT, RPT)])
    plsc.subcore_barrier()
    lane = lax.broadcasted_iota(jnp.int32, (16,), 0)
    zero16 = lane // 16
    jcols = [zero16 + j for j in range(16)]
    m1 = lane < 1

    def gathers(i, h2r, gsem):
        pltpu.async_copy(hs2_h.at[sidxa.at[i]], h2r, gsem)

    def drain(i, h2r, gsem):
        pltpu.make_async_copy(hs2_h.at[sidxa.at[i]], h2r, gsem).wait()

    def compute(i, h2r, outr):
        def grp(g, c2):
            s16 = sidxa[i, pl.ds(g * 16, 16)]
            d16 = didxa[i, pl.ds(g * 16, 16)]
            asg, _ = plsc.unpack(
                plsc.bitcast(plsc.load_gather(aspk, [s16]), jnp.bfloat16),
                format=plsc.PackFormat.INTERLEAVED)
            _, adg = plsc.unpack(
                plsc.bitcast(plsc.load_gather(aspk, [d16]), jnp.bfloat16),
                format=plsc.PackFormat.INTERLEAVED)
            ev = asg + adg
            ev = jnp.maximum(ev, 0.2 * ev)
            ex16 = jnp.exp(ev)
            exbuf[...] = ex16
            for j in range(16):
                e = g * 16 + j
                exb = plsc.load_gather(exbuf, [jcols[j]])
                dv = plsc.load_gather(didxa, [zero16 + i, zero16 + e])
                plsc.addupdate_scatter(s2loc, [dv], exb, mask=m1)
                for k in range(4):
                    hv = h2r[e, pl.ds(32 * k, 32)]
                    av, bv = plsc.unpack(
                        hv, format=plsc.PackFormat.INTERLEAVED)
                    outr[e, pl.ds(32 * k, 16)] = av * exb
                    outr[e, pl.ds(32 * k + 16, 16)] = bv * exb
            return c2

        lax.fori_loop(0, B2 // 16, grp, 0)

    def scatter(i, outr):
        pltpu.sync_copy(outr, accm.at[didxa.at[i]], add=True)

    def phase(p, pcarry):
        row0 = wid * NBLK2 + p * PBLK2
        pltpu.sync_copy(src_h.at[pl.ds(row0, PBLK2)], sidxa)
        pltpu.sync_copy(dst_h.at[pl.ds(row0, PBLK2)], didxa)
        gathers(0, h2r0, gs0)

        def body(j, carry):
            i0 = 2 * j
            i1 = i0 + 1
            gathers(i1, h2r1, gs1)
            drain(i0, h2r0, gs0)
            compute(i0, h2r0, outr0)

            @pl.when(i0 + 2 < PBLK2)
            def _():
                gathers(i0 + 2, h2r0, gs0)

            scatter(i0, outr0)
            drain(i1, h2r1, gs1)
            compute(i1, h2r1, outr1)

            @pl.when(i1 + 2 < PBLK2)
            def _():
                gathers(i1 + 2, h2r1, gs1)

            scatter(i1, outr1)
            return carry

        lax.fori_loop(0, PBLK2 // 2, body, 0)
        return pcarry

    lax.fori_loop(0, NBLK2 // PBLK2, phase, 0)
    plsc.subcore_barrier()
    pltpu.sync_copy(accm.at[pl.ds(sid * RPT, RPT)],
                    msg_o.at[cid].at[pl.ds(sid * RPT, RPT)])
    pltpu.sync_copy(s2loc, s_o.at[cid].at[sid])


def kernel(x, edge_index, W1, a_src1, a_dst1, b1, W2, a_src2, a_dst2, b2):
    f32 = jnp.float32
    i32 = jnp.int32

    e_real = edge_index.shape[1]
    e_pad = NW * NBLK1 * B
    assert e_pad >= e_real and NBLK1 * B == NBLK2 * B2
    pad = jnp.full((e_pad - e_real,), N, dtype=i32)
    src = jnp.concatenate([edge_index[0].astype(i32), pad])
    dst = jnp.concatenate([edge_index[1].astype(i32), pad])
    src2d = src.reshape(NW * NBLK1, B)
    dst2d = dst.reshape(NW * NBLK1, B)
    src2d64 = src.reshape(NW * NBLK2, B2)
    dst2d64 = dst.reshape(NW * NBLK2, B2)

    k64 = jnp.arange(HEADS * HID)
    as64 = jnp.zeros((HEADS * HID, HEADS), f32).at[k64, k64 // HID].set(
        a_src1.reshape(-1))
    ad64 = jnp.zeros((HEADS * HID, HEADS), f32).at[k64, k64 // HID].set(
        a_dst1.reshape(-1))
    r8 = jnp.zeros((HEADS, HEADS * HID), f32).at[k64 // HID, k64].set(1.0)
    j64 = jnp.arange(64)
    perm = 32 * (j64 // 32) + (j64 % 2) * 16 + (j64 % 32) // 2
    asport = jnp.zeros((D_IN, 32), f32).at[:, 2 * jnp.arange(8)].set(
        W1 @ as64)
    w1cat = jnp.concatenate([W1[:, perm], asport], axis=1)
    w1ad = jnp.concatenate(
        [W1 @ ad64, jnp.zeros((D_IN, 8), f32)], axis=1)
    waa = jnp.concatenate(
        [W2 @ a_src2.reshape(-1, 1), W2 @ a_dst2.reshape(-1, 1),
         jnp.zeros((HEADS * HID, 14), f32)], axis=1)
    ones32 = jnp.ones((NW, 1), f32)
    j128 = jnp.arange(128)
    perm128 = 32 * (j128 // 32) + (j128 % 2) * 16 + (j128 % 32) // 2
    w2p = W2[:, perm128]

    x_pad = jnp.zeros((NPAD, D_IN), f32).at[:N].set(x)

    hs1, ad1, init1 = pl.pallas_call(
        _tc_a,
        out_shape=(jax.ShapeDtypeStruct((NPAD, W1ROW), jnp.bfloat16),
                   jax.ShapeDtypeStruct((NPAD, A1ROW), f32),
                   jax.ShapeDtypeStruct((NPAD, 72), f32)),
    )(x_pad, w1cat, w1ad, W1, W1 @ as64, r8)

    mesh = plsc.VectorSubcoreMesh(core_axis_name="c", subcore_axis_name="s")
    sc_params = pltpu.CompilerParams(
        use_tc_tiling_on_sc=False, needs_layout_passes=False)
    msg1 = pl.kernel(
        _sc_edge1,
        out_type=jax.ShapeDtypeStruct((NC, NPAD, 72), f32),
        mesh=mesh,
        compiler_params=sc_params,
        scratch_types=[
            pltpu.VMEM((PBLK1, B), i32),
            pltpu.VMEM((PBLK1, B), i32),
            pltpu.VMEM((B, W1ROW), jnp.bfloat16),
            pltpu.VMEM((B, W1ROW), jnp.bfloat16),
            pltpu.VMEM((B, A1ROW), f32),
            pltpu.VMEM((B, A1ROW), f32),
            pltpu.VMEM((B, 72), f32),
            pltpu.VMEM((B, 72), f32),
            pltpu.VMEM((16,), f32),
            pltpu.VMEM_SHARED((NPAD, 72), f32),
            pltpu.SemaphoreType.DMA,
            pltpu.SemaphoreType.DMA,
        ],
    )(src2d, dst2d, hs1, ad1, init1)

    hs2, aa, init2 = pl.pallas_call(
        _tc_b,
        out_shape=(jax.ShapeDtypeStruct((NPAD, D_OUT), jnp.bfloat16),
                   jax.ShapeDtypeStruct((NPAD, 16), f32),
                   jax.ShapeDtypeStruct((NPAD, D_OUT), f32)),
    )(msg1[0], msg1[1], r8, w2p, W2, waa, b1.reshape(1, -1))

    as2u = lax.bitcast_convert_type(
        aa[:, 0].astype(jnp.bfloat16), jnp.uint16).astype(jnp.uint32)
    ad2u = lax.bitcast_convert_type(
        aa[:, 1].astype(jnp.bfloat16), jnp.uint16).astype(jnp.uint32)
    aspk = lax.bitcast_convert_type(
        (ad2u << 16) | as2u, jnp.int32).reshape(NPAD)

    zs2 = jnp.zeros((NPAD,), f32)
    msg2, s2 = pl.kernel(
        _sc_edge2,
        out_type=(jax.ShapeDtypeStruct((NC, NPAD, D_OUT), f32),
                  jax.ShapeDtypeStruct((NC, NS, NPAD), f32)),
        mesh=mesh,
        compiler_params=sc_params,
        scratch_types=[
            pltpu.VMEM((PBLK2, B2), i32),
            pltpu.VMEM((PBLK2, B2), i32),
            pltpu.VMEM((B2, D_OUT), jnp.bfloat16),
            pltpu.VMEM((B2, D_OUT), jnp.bfloat16),
            pltpu.VMEM((B2, D_OUT), f32),
            pltpu.VMEM((B2, D_OUT), f32),
            pltpu.VMEM((NPAD,), i32),
            pltpu.VMEM((16,), f32),
            pltpu.VMEM((NPAD,), f32),
            pltpu.VMEM_SHARED((NPAD, D_OUT), f32),
            pltpu.SemaphoreType.DMA,
            pltpu.SemaphoreType.DMA,
        ],
    )(src2d64, dst2d64, hs2, aspk, init2, zs2)

    out = pl.pallas_call(
        _tc_c,
        out_shape=jax.ShapeDtypeStruct((NPAD, D_OUT), f32),
    )(msg2[0], msg2[1], s2.reshape(NW, NPAD), ones32, aa,
      b2.reshape(1, -1))
    return out[:N]


# speedup vs baseline: 55.3610x; 1.0001x over previous
import jax
import jax.numpy as jnp
from jax import lax
from jax.experimental import pallas as pl
from jax.experimental.pallas import tpu as pltpu
from jax.experimental.pallas import tpu_sc as plsc

N = 10000
D_IN = 128
HID = 8
HEADS = 8
D_OUT = 128

NPAD = 10240
NC, NS = 2, 16
NW = NC * NS
RPT = NPAD // NS
B = 128
B2 = 64
PBLK1 = 40
PBLK2 = 20
NBLK1 = 2 * PBLK1
NBLK2 = 8 * PBLK2

W1ROW = 96
A1ROW = 16


def _tc_a(x_ref, w_ref, wa_ref, w1_ref, w1as_ref, r8_ref,
          hs_ref, ad_ref, init_ref):
    x = x_ref[...]
    hs_ref[...] = jnp.dot(
        x, w_ref[...], preferred_element_type=jnp.float32
    ).astype(jnp.bfloat16)
    ad = jnp.dot(x, wa_ref[...], preferred_element_type=jnp.float32)
    ad_ref[...] = ad
    h1 = jnp.dot(x, w1_ref[...], preferred_element_type=jnp.float32)
    ev = (jnp.dot(x, w1as_ref[...], preferred_element_type=jnp.float32)
          + ad[:, 0:8])
    ev = jnp.maximum(ev, 0.2 * ev)
    exs = 0.5 * jnp.exp(ev)
    exrep = jnp.dot(exs, r8_ref[...], preferred_element_type=jnp.float32)
    init_ref[:, 0:64] = h1 * exrep
    init_ref[:, 64:72] = exs


def _tc_b(m0_ref, m1_ref, r8_ref, w2p_ref, w2_ref, waa_ref, b1_ref,
          hs2_ref, aa_ref, init_ref):
    acc = m0_ref[...] + m1_ref[...]
    srep = jnp.dot(acc[:, 64:72], r8_ref[...],
                   preferred_element_type=jnp.float32)
    g = acc[:, 0:64] / (srep + 1e-16) + b1_ref[...]
    g = jnp.where(g > 0, g, jnp.exp(g) - 1.0)
    hs2_ref[...] = jnp.dot(
        g, w2p_ref[...], preferred_element_type=jnp.float32
    ).astype(jnp.bfloat16)
    aav = jnp.dot(g, waa_ref[...], preferred_element_type=jnp.float32)
    ev = aav[:, 0:1] + aav[:, 1:2]
    ev = jnp.maximum(ev, 0.2 * ev)
    exs2 = jnp.exp(ev)
    aa_ref[...] = jnp.concatenate(
        [aav[:, 0:2], exs2, aav[:, 3:16]], axis=1)
    h2n = jnp.dot(g, w2_ref[...], preferred_element_type=jnp.float32)
    init_ref[...] = h2n * (0.5 * exs2)


def _tc_c(m0_ref, m1_ref, sp_ref, ones_ref, aa_ref, b2_ref, out_ref):
    s = lax.dot_general(sp_ref[...], ones_ref[...],
                        (((0,), (0,)), ((), ())),
                        preferred_element_type=jnp.float32)
    s = s + aa_ref[:, 2:3]
    v = (m0_ref[...] + m1_ref[...]) / (s + 1e-16) + b2_ref[...]
    m = jnp.max(v, axis=1, keepdims=True)
    u = v - m
    lse = jnp.log(jnp.sum(jnp.exp(u), axis=1, keepdims=True))
    out_ref[...] = u - lse


def _sc_edge1(src_h, dst_h, hs_h, ad_h, zm_h, msg_o,
              sidxa, didxa, hsr0, hsr1, adr0, adr1, outr0, outr1,
              exbuf, accm, gs0, gs1):
    cid = lax.axis_index("c")
    sid = lax.axis_index("s")
    wid = sid * NC + cid
    pltpu.sync_copy(zm_h.at[pl.ds(sid * RPT, RPT)],
                    accm.at[pl.ds(sid * RPT, RPT)])
    plsc.subcore_barrier()
    lane = lax.broadcasted_iota(jnp.int32, (16,), 0)
    half = lane // 8
    zero16 = lane // 16
    cols = [half + 2 * k for k in range(4)]
    lane7 = jnp.bitwise_and(lane, 7)
    m8 = lane < 8

    def gathers(i, hsr, adr, gsem):
        a = pltpu.async_copy(hs_h.at[sidxa.at[i]], hsr, gsem)
        b = pltpu.async_copy(ad_h.at[didxa.at[i]], adr, gsem)
        return a, b

    def drain(i, hsr, adr, gsem):
        pltpu.make_async_copy(hs_h.at[sidxa.at[i]], hsr, gsem).wait()
        pltpu.make_async_copy(ad_h.at[didxa.at[i]], adr, gsem).wait()

    def compute(hsr, adr, outr):
        def edge(e, c2):
            as32 = hsr[e, pl.ds(64, 32)]
            asv, _ = plsc.unpack(as32, format=plsc.PackFormat.INTERLEAVED)
            adv = adr[e, :]
            ev = asv + adv
            ev = jnp.maximum(ev, 0.2 * ev)
            ex = jnp.exp(ev)
            exbuf[...] = ex
            rowi = zero16 + e
            plsc.store_scatter(outr, [rowi, 64 + lane7], ex, mask=m8)
            for k in range(2):
                hv = hsr[e, pl.ds(32 * k, 32)]
                av, bv = plsc.unpack(hv, format=plsc.PackFormat.INTERLEAVED)
                exba = plsc.load_gather(exbuf, [cols[2 * k]])
                exbb = plsc.load_gather(exbuf, [cols[2 * k + 1]])
                outr[e, pl.ds(32 * k, 16)] = av * exba
                outr[e, pl.ds(32 * k + 16, 16)] = bv * exbb
            return c2

        lax.fori_loop(0, B, edge, 0, unroll=2)

    def scatter(i, outr):
        pltpu.sync_copy(outr, accm.at[didxa.at[i]], add=True)

    def phase(p, pcarry):
        row0 = wid * NBLK1 + p * PBLK1
        pltpu.sync_copy(src_h.at[pl.ds(row0, PBLK1)], sidxa)
        pltpu.sync_copy(dst_h.at[pl.ds(row0, PBLK1)], didxa)
        gathers(0, hsr0, adr0, gs0)

        def body(j, carry):
            i0 = 2 * j
            i1 = i0 + 1
            c1a, c1b = gathers(i1, hsr1, adr1, gs1)
            drain(i0, hsr0, adr0, gs0)
            compute(hsr0, adr0, outr0)

            @pl.when(i0 + 2 < PBLK1)
            def _():
                gathers(i0 + 2, hsr0, adr0, gs0)

            scatter(i0, outr0)
            c1a.wait()
            c1b.wait()
            compute(hsr1, adr1, outr1)

            @pl.when(i1 + 2 < PBLK1)
            def _():
                gathers(i1 + 2, hsr1, adr1, gs1)

            scatter(i1, outr1)
            return carry

        lax.fori_loop(0, PBLK1 // 2, body, 0)
        return pcarry

    lax.fori_loop(0, NBLK1 // PBLK1, phase, 0)
    plsc.subcore_barrier()
    pltpu.sync_copy(accm.at[pl.ds(sid * RPT, RPT)],
                    msg_o.at[cid].at[pl.ds(sid * RPT, RPT)])


def _sc_edge2(src_h, dst_h, hs2_h, aspk_h, zm_h, zs_h, msg_o, s_o,
              sidxa, didxa, h2r0, h2r1, outr0, outr1, aspk, exbuf,
              s2loc, accm, gs0, gs1):
    cid = lax.axis_index("c")
    sid = lax.axis_index("s")
    wid = sid * NC + cid
    pltpu.sync_copy(aspk_h, aspk)
    pltpu.sync_copy(zs_h, s2loc)
    pltpu.sync_copy(zm_h.at[pl.ds(sid * RPT, RPT)],
                    accm.at[pl.ds(sid * RPT, RPT)])
    plsc.subcore_barrier()
    lane = lax.broadcasted_iota(jnp.int32, (16,), 0)
    zero16 = lane // 16
    jcols = [zero16 + j for j in range(16)]
    m1 = lane < 1

    def gathers(i, h2r, gsem):
        pltpu.async_copy(hs2_h.at[sidxa.at[i]], h2r, gsem)

    def drain(i, h2r, gsem):
        pltpu.make_async_copy(hs2_h.at[sidxa.at[i]], h2r, gsem).wait()

    def compute(i, h2r, outr):
        def grp(g, c2):
            s16 = sidxa[i, pl.ds(g * 16, 16)]
            d16 = didxa[i, pl.ds(g * 16, 16)]
            asg, _ = plsc.unpack(
                plsc.bitcast(plsc.load_gather(aspk, [s16]), jnp.bfloat16),
                format=plsc.PackFormat.INTERLEAVED)
            _, adg = plsc.unpack(
                plsc.bitcast(plsc.load_gather(aspk, [d16]), jnp.bfloat16),
                format=plsc.PackFormat.INTERLEAVED)
            ev = asg + adg
            ev = jnp.maximum(ev, 0.2 * ev)
            ex16 = jnp.exp(ev)
            exbuf[...] = ex16
            for j in range(16):
                e = g * 16 + j
                exb = plsc.load_gather(exbuf, [jcols[j]])
                dv = plsc.load_gather(didxa, [zero16 + i, zero16 + e])
                plsc.addupdate_scatter(s2loc, [dv], exb, mask=m1)
                for k in range(4):
                    hv = h2r[e, pl.ds(32 * k, 32)]
                    av, bv = plsc.unpack(
                        hv, format=plsc.PackFormat.INTERLEAVED)
                    outr[e, pl.ds(32 * k, 16)] = av * exb
                    outr[e, pl.ds(32 * k + 16, 16)] = bv * exb
            return c2

        lax.fori_loop(0, B2 // 16, grp, 0)

    def scatter(i, outr):
        pltpu.sync_copy(outr, accm.at[didxa.at[i]], add=True)

    def phase(p, pcarry):
        row0 = wid * NBLK2 + p * PBLK2
        pltpu.sync_copy(src_h.at[pl.ds(row0, PBLK2)], sidxa)
        pltpu.sync_copy(dst_h.at[pl.ds(row0, PBLK2)], didxa)
        gathers(0, h2r0, gs0)

        def body(j, carry):
            i0 = 2 * j
            i1 = i0 + 1
            gathers(i1, h2r1, gs1)
            drain(i0, h2r0, gs0)
            compute(i0, h2r0, outr0)

            @pl.when(i0 + 2 < PBLK2)
            def _():
                gathers(i0 + 2, h2r0, gs0)

            scatter(i0, outr0)
            drain(i1, h2r1, gs1)
            compute(i1, h2r1, outr1)

            @pl.when(i1 + 2 < PBLK2)
            def _():
                gathers(i1 + 2, h2r1, gs1)

            scatter(i1, outr1)
            return carry

        lax.fori_loop(0, PBLK2 // 2, body, 0)
        return pcarry

    lax.fori_loop(0, NBLK2 // PBLK2, phase, 0)
    plsc.subcore_barrier()
    pltpu.sync_copy(accm.at[pl.ds(sid * RPT, RPT)],
                    msg_o.at[cid].at[pl.ds(sid * RPT, RPT)])
    pltpu.sync_copy(s2loc, s_o.at[cid].at[sid])


def kernel(x, edge_index, W1, a_src1, a_dst1, b1, W2, a_src2, a_dst2, b2):
    f32 = jnp.float32
    i32 = jnp.int32

    e_real = edge_index.shape[1]
    e_pad = NW * NBLK1 * B
    assert e_pad >= e_real and NBLK1 * B == NBLK2 * B2
    pad = jnp.full((e_pad - e_real,), N, dtype=i32)
    src = jnp.concatenate([edge_index[0].astype(i32), pad])
    dst = jnp.concatenate([edge_index[1].astype(i32), pad])
    src2d = src.reshape(NW * NBLK1, B)
    dst2d = dst.reshape(NW * NBLK1, B)
    src2d64 = src.reshape(NW * NBLK2, B2)
    dst2d64 = dst.reshape(NW * NBLK2, B2)

    k64 = jnp.arange(HEADS * HID)
    as64 = jnp.zeros((HEADS * HID, HEADS), f32).at[k64, k64 // HID].set(
        a_src1.reshape(-1))
    ad64 = jnp.zeros((HEADS * HID, HEADS), f32).at[k64, k64 // HID].set(
        a_dst1.reshape(-1))
    r8 = jnp.zeros((HEADS, HEADS * HID), f32).at[k64 // HID, k64].set(1.0)
    j64 = jnp.arange(64)
    perm = 32 * (j64 // 32) + (j64 % 2) * 16 + (j64 % 32) // 2
    asport = jnp.zeros((D_IN, 32), f32).at[:, 2 * jnp.arange(8)].set(
        W1 @ as64)
    w1cat = jnp.concatenate([W1[:, perm], asport], axis=1)
    w1ad = jnp.concatenate(
        [W1 @ ad64, jnp.zeros((D_IN, 8), f32)], axis=1)
    waa = jnp.concatenate(
        [W2 @ a_src2.reshape(-1, 1), W2 @ a_dst2.reshape(-1, 1),
         jnp.zeros((HEADS * HID, 14), f32)], axis=1)
    ones32 = jnp.ones((NW, 1), f32)
    j128 = jnp.arange(128)
    perm128 = 32 * (j128 // 32) + (j128 % 2) * 16 + (j128 % 32) // 2
    w2p = W2[:, perm128]

    x_pad = jnp.zeros((NPAD, D_IN), f32).at[:N].set(x)

    hs1, ad1, init1 = pl.pallas_call(
        _tc_a,
        out_shape=(jax.ShapeDtypeStruct((NPAD, W1ROW), jnp.bfloat16),
                   jax.ShapeDtypeStruct((NPAD, A1ROW), f32),
                   jax.ShapeDtypeStruct((NPAD, 72), f32)),
    )(x_pad, w1cat, w1ad, W1, W1 @ as64, r8)

    mesh = plsc.VectorSubcoreMesh(core_axis_name="c", subcore_axis_name="s")
    sc_params = pltpu.CompilerParams(
        use_tc_tiling_on_sc=False, needs_layout_passes=False)
    msg1 = pl.kernel(
        _sc_edge1,
        out_type=jax.ShapeDtypeStruct((NC, NPAD, 72), f32),
        mesh=mesh,
        compiler_params=sc_params,
        scratch_types=[
            pltpu.VMEM((PBLK1, B), i32),
            pltpu.VMEM((PBLK1, B), i32),
            pltpu.VMEM((B, W1ROW), jnp.bfloat16),
            pltpu.VMEM((B, W1ROW), jnp.bfloat16),
            pltpu.VMEM((B, A1ROW), f32),
            pltpu.VMEM((B, A1ROW), f32),
            pltpu.VMEM((B, 72), f32),
            pltpu.VMEM((B, 72), f32),
            pltpu.VMEM((16,), f32),
            pltpu.VMEM_SHARED((NPAD, 72), f32),
            pltpu.SemaphoreType.DMA,
            pltpu.SemaphoreType.DMA,
        ],
    )(src2d, dst2d, hs1, ad1, init1)

    hs2, aa, init2 = pl.pallas_call(
        _tc_b,
        out_shape=(jax.ShapeDtypeStruct((NPAD, D_OUT), jnp.bfloat16),
                   jax.ShapeDtypeStruct((NPAD, 16), f32),
                   jax.ShapeDtypeStruct((NPAD, D_OUT), f32)),
    )(msg1[0], msg1[1], r8, w2p, W2, waa, b1.reshape(1, -1))

    as2u = lax.bitcast_convert_type(
        aa[:, 0].astype(jnp.bfloat16), jnp.uint16).astype(jnp.uint32)
    ad2u = lax.bitcast_convert_type(
        aa[:, 1].astype(jnp.bfloat16), jnp.uint16).astype(jnp.uint32)
    aspk = lax.bitcast_convert_type(
        (ad2u << 16) | as2u, jnp.int32).reshape(NPAD)

    zs2 = jnp.zeros((NPAD,), f32)
    msg2, s2 = pl.kernel(
        _sc_edge2,
        out_type=(jax.ShapeDtypeStruct((NC, NPAD, D_OUT), f32),
                  jax.ShapeDtypeStruct((NC, NS, NPAD), f32)),
        mesh=mesh,
        compiler_params=sc_params,
        scratch_types=[
            pltpu.VMEM((PBLK2, B2), i32),
            pltpu.VMEM((PBLK2, B2), i32),
            pltpu.VMEM((B2, D_OUT), jnp.bfloat16),
            pltpu.VMEM((B2, D_OUT), jnp.bfloat16),
            pltpu.VMEM((B2, D_OUT), f32),
            pltpu.VMEM((B2, D_OUT), f32),
            pltpu.VMEM((NPAD,), i32),
            pltpu.VMEM((16,), f32),
            pltpu.VMEM((NPAD,), f32),
            pltpu.VMEM_SHARED((NPAD, D_OUT), f32),
            pltpu.SemaphoreType.DMA,
            pltpu.SemaphoreType.DMA,
        ],
    )(src2d64, dst2d64, hs2, aspk, init2, zs2)

    out = pl.pallas_call(
        _tc_c,
        out_shape=jax.ShapeDtypeStruct((NPAD, D_OUT), f32),
    )(msg2[0], msg2[1], s2.reshape(NW, NPAD), ones32, aa,
      b2.reshape(1, -1))
    return out[:N]
